# baseline (speedup 1.0000x reference)
import functools

import jax
import jax.numpy as jnp
from jax import lax
from jax.experimental import pallas as pl
from jax.experimental.pallas import tpu as pltpu
from jax.experimental.pallas import tpu_sc as plsc

NC, NS = 2, 16
NW = NC * NS
N0, E, H, LAT = 10000, 640000, 128, 16
NPAD0 = 10240
PT = E // NW
ECH = 2560
NECH = E // ECH
PTP = 20608
NCH = PTP // 128
NSUB = E // 128
NBLK8 = E // (128 * 8)

K1, K2, K3 = 5000, 2500, 1250
KP1, KP2, KP3 = 5120, 2560, 1280
KD1, KD2, KD3 = 5248, 2688, 1408
CDUMP = 3 * NW * PTP

_f32 = jnp.float32
_i32 = jnp.int32


def _sc_mesh():
    return plsc.VectorSubcoreMesh(core_axis_name="c", subcore_axis_name="s")


def _wid():
    return lax.axis_index("s") * NC + lax.axis_index("c")


def _edge_prep_body(src_hbm, dst_hbm, one_hbm, z640_hbm,
                    deg_out, degdec_out, comps_out, compd_out, cnt_out,
                    sbuf, dbuf, pidx, vsb, vdb, si128, di128, w128, one128,
                    cntv, degsh, ddsh):
    c = lax.axis_index("c")
    s = lax.axis_index("s")
    wid = _wid()

    pltpu.sync_copy(one_hbm, one128)
    pltpu.sync_copy(z640_hbm, degsh.at[pl.ds(s * 640, 640)])

    @pl.when(s < KD3 // 128)
    def _():
        pltpu.sync_copy(z640_hbm.at[pl.ds(0, 128)], ddsh.at[pl.ds(s * 128, 128)])

    plsc.subcore_barrier()

    def deg_body(j, carry):
        sub = wid + NW * j

        @pl.when(sub < NSUB)
        def _():
            base = sub * 128
            pltpu.sync_copy(src_hbm.at[pl.ds(base, 128)], si128)
            pltpu.sync_copy(dst_hbm.at[pl.ds(base, 128)], di128)
            pltpu.sync_copy(one128, degsh.at[di128], add=True)
            for v in range(8):
                sv = si128[pl.ds(v * 16, 16)]
                dv = di128[pl.ds(v * 16, 16)]
                mA = jnp.maximum(sv, dv) < K3
                w128[pl.ds(v * 16, 16)] = jnp.where(mA, 1.0, 0.0).astype(_f32)
                di128[pl.ds(v * 16, 16)] = jnp.where(mA, dv, KP3)
            pltpu.sync_copy(w128, ddsh.at[di128], add=True)

        return carry

    lax.fori_loop(0, (NSUB + NW - 1) // NW, deg_body, 0)

    cuts = [(0, K3), (K3, K2), (K2, K1)]
    offs = []
    lanes16 = lax.iota(_i32, 16)
    for b, (lo, hi) in enumerate(cuts):
        seg = (b * NW + wid) * PTP

        def chunk_body(i, off, lo=lo, hi=hi, seg=seg):
            ch = wid + NW * i

            def do_chunk(off_in):
                pltpu.sync_copy(src_hbm.at[pl.ds(ch * ECH, ECH)], sbuf)
                pltpu.sync_copy(dst_hbm.at[pl.ds(ch * ECH, ECH)], dbuf)

                def grp_body(g, off2):
                    for v in range(8):
                        sv = sbuf[pl.ds(g * 128 + v * 16, 16)]
                        dv = dbuf[pl.ds(g * 128 + v * 16, 16)]
                        mx = jnp.maximum(sv, dv)
                        mb = (mx >= lo) & (mx < hi)
                        cs = plsc.cumsum(mb.astype(_i32))
                        pos = jnp.where(mb, seg + off2 + cs - 1,
                                        CDUMP + v * 16 + lanes16)
                        pidx[pl.ds(v * 16, 16)] = pos
                        vsb[pl.ds(v * 16, 16)] = sv
                        vdb[pl.ds(v * 16, 16)] = dv
                        off2 = off2 + cs[15]
                    pltpu.sync_copy(vsb, comps_out.at[pidx])
                    pltpu.sync_copy(vdb, compd_out.at[pidx])
                    return off2

                return lax.fori_loop(0, ECH // 128, grp_body, off_in)

            return lax.cond(ch < NECH, do_chunk, lambda o: o, off)

        off_b = lax.fori_loop(0, (NECH + NW - 1) // NW, chunk_body,
                              jnp.int32(0))
        offs.append(off_b)

    lanes = lax.iota(_i32, 16)
    cvec = jnp.zeros((16,), _i32)
    for b, off_b in enumerate(offs):
        cvec = jnp.where(lanes == b, jnp.zeros((16,), _i32) + off_b, cvec)
    cntv[pl.ds(0, 16)] = cvec
    for t in range(1, 8):
        cntv[pl.ds(t * 16, 16)] = jnp.zeros((16,), _i32)
    pltpu.sync_copy(cntv, cnt_out.at[pl.ds(wid * 128, 128)])

    plsc.subcore_barrier()

    @pl.when(s == 0)
    def _():
        pltpu.sync_copy(degsh, deg_out.at[pl.ds(c * NPAD0, NPAD0)])
        pltpu.sync_copy(ddsh, degdec_out.at[pl.ds(c * KD3, KD3)])


def _edge_prep(src, dst):
    ones = jnp.ones((128,), _f32)
    z640 = jnp.zeros((640,), _f32)
    kern = pl.kernel(
        _edge_prep_body,
        out_type=(
            jax.ShapeDtypeStruct((NC * NPAD0,), _f32),
            jax.ShapeDtypeStruct((NC * KD3,), _f32),
            jax.ShapeDtypeStruct((CDUMP + 128,), _i32),
            jax.ShapeDtypeStruct((CDUMP + 128,), _i32),
            jax.ShapeDtypeStruct((NW * 128,), _i32),
        ),
        mesh=_sc_mesh(),
        compiler_params=pltpu.CompilerParams(needs_layout_passes=False),
        scratch_types=[
            pltpu.VMEM((ECH,), _i32), pltpu.VMEM((ECH,), _i32),
            pltpu.VMEM((128,), _i32), pltpu.VMEM((128,), _i32),
            pltpu.VMEM((128,), _i32),
            pltpu.VMEM((128,), _i32), pltpu.VMEM((128,), _i32),
            pltpu.VMEM((128,), _f32), pltpu.VMEM((128,), _f32),
            pltpu.VMEM((128,), _i32),
            pltpu.VMEM_SHARED((NPAD0,), _f32),
            pltpu.VMEM_SHARED((KD3,), _f32),
        ],
        name="edge_prep",
    )
    return kern(src, dst, ones, z640)


def _gcn0_agg_body(y_hbm, src_hbm, dst_hbm, z640_hbm, agg_out,
                   si128, di128, gidx, sidx, gbuf, aggsh):
    c = lax.axis_index("c")
    s = lax.axis_index("s")
    wid = _wid()

    for q in range(4):
        pltpu.sync_copy(z640_hbm, aggsh.at[pl.ds(s * 2560 + q * 640, 640)])
    plsc.subcore_barrier()

    def blk_body(j, carry):
        sub = wid + NW * j

        @pl.when(sub < NSUB)
        def _():
            base = sub * 128
            pltpu.sync_copy(src_hbm.at[pl.ds(base, 128)], si128)
            pltpu.sync_copy(dst_hbm.at[pl.ds(base, 128)], di128)
            for cc in range(4):
                for v in range(8):
                    sv = si128[pl.ds(v * 16, 16)]
                    dv = di128[pl.ds(v * 16, 16)]
                    gidx[pl.ds(v * 16, 16)] = sv * 4 + cc
                    sidx[pl.ds(v * 16, 16)] = dv * 4 + cc
                pltpu.sync_copy(y_hbm.at[gidx], gbuf)
                pltpu.sync_copy(gbuf, aggsh.at[sidx], add=True)

        return carry

    lax.fori_loop(0, (NSUB + NW - 1) // NW, blk_body, 0)
    plsc.subcore_barrier()
    pltpu.sync_copy(aggsh.at[pl.ds(s * 2560, 2560)],
                    agg_out.at[pl.ds(c * NPAD0 * 4 + s * 2560, 2560)])


def _gcn0_agg(y, src, dst):
    z640 = jnp.zeros((640,), _f32)
    kern = pl.kernel(
        _gcn0_agg_body,
        out_type=jax.ShapeDtypeStruct((NC * NPAD0 * 4,), _f32),
        mesh=_sc_mesh(),
        compiler_params=pltpu.CompilerParams(needs_layout_passes=False),
        scratch_types=[
            pltpu.VMEM((128,), _i32), pltpu.VMEM((128,), _i32),
            pltpu.VMEM((128,), _i32), pltpu.VMEM((128,), _i32),
            pltpu.VMEM((128,), _f32),
            pltpu.VMEM_SHARED((NPAD0 * 4,), _f32),
        ],
        name="gcn0_agg",
    )
    return kern(y.reshape(NPAD0 * 4), src, dst, z640)


def _gat_edge_body(kpad, kd, buckets,
                   comps, compd, cnt_hbm, als_hbm, ald_hbm, m_hbm, h_hbm,
                   znum_hbm, zden_hbm,
                   num_out, den_out,
                   alsv, aldv, mv, sbufw, dbufw, wbuf, rowbuf,
                   cntv, hsh, numsh, densh):
    c = lax.axis_index("c")
    s = lax.axis_index("s")
    wid = _wid()
    rows_h = kpad // 16
    rows_t = kd // 16
    dump = kpad

    nchd = kd // 128
    pltpu.sync_copy(als_hbm, alsv)
    pltpu.sync_copy(ald_hbm, aldv)
    pltpu.sync_copy(m_hbm, mv)
    pltpu.sync_copy(cnt_hbm.at[pl.ds(wid * 128, 128)], cntv)
    pltpu.sync_copy(h_hbm.at[pl.ds(s * rows_h, rows_h)],
                    hsh.at[pl.ds(s * rows_h, rows_h)])
    pltpu.sync_copy(znum_hbm, numsh.at[pl.ds(s * rows_t, rows_t)])
    for i in range((nchd + NS - 1) // NS):
        chunk = s + NS * i

        @pl.when(chunk < nchd)
        def _(chunk=chunk):
            pltpu.sync_copy(zden_hbm, densh.at[pl.ds(chunk * 128, 128)])

    plsc.subcore_barrier()

    cntvec = cntv[pl.ds(0, 16)]
    for b in buckets:
        cb = cntvec[b]
        segbase = (b * NW + wid) * PTP

        def chunk_body(j, carry, cb=cb, segbase=segbase):
            @pl.when(j * 128 < cb)
            def _():
                pltpu.sync_copy(comps.at[pl.ds(segbase + j * 128, 128)], sbufw)
                pltpu.sync_copy(compd.at[pl.ds(segbase + j * 128, 128)], dbufw)
                for v in range(8):
                    sv = sbufw[pl.ds(v * 16, 16)]
                    dv = dbufw[pl.ds(v * 16, 16)]
                    lanepos = j * 128 + v * 16 + lax.iota(_i32, 16)
                    mvld = lanepos < cb
                    s_c = jnp.where(mvld, sv, 0)
                    d_t = jnp.where(mvld, dv, 0)
                    a1 = plsc.load_gather(alsv, [s_c])
                    a2 = plsc.load_gather(aldv, [d_t])
                    mm = plsc.load_gather(mv, [d_t])
                    e = a1 + a2
                    e = jnp.where(e > 0, e, 0.2 * e)
                    w = jnp.where(mvld, jnp.exp(e - mm), 0.0)
                    wbuf[pl.ds(v * 16, 16)] = w
                    sbufw[pl.ds(v * 16, 16)] = s_c
                    dbufw[pl.ds(v * 16, 16)] = jnp.where(mvld, dv, dump)
                pltpu.sync_copy(wbuf.at[pl.ds(0, 128)], densh.at[dbufw],
                                add=True)
                pltpu.sync_copy(hsh.at[sbufw], rowbuf)

                def scale_row(r, carry2):
                    wr = wbuf[pl.ds(r, 16)][0]
                    for cc in range(8):
                        rowbuf[r, pl.ds(cc * 16, 16)] = (
                            rowbuf[r, pl.ds(cc * 16, 16)] * wr)
                    return carry2

                lax.fori_loop(0, 128, scale_row, 0)
                pltpu.sync_copy(rowbuf, numsh.at[dbufw], add=True)

            return carry

        lax.fori_loop(0, NCH, chunk_body, 0)

    plsc.subcore_barrier()
    pltpu.sync_copy(numsh.at[pl.ds(s * rows_t, rows_t)],
                    num_out.at[pl.ds(c * kd + s * rows_t, rows_t)])
    for i in range((nchd + NS - 1) // NS):
        chunk = s + NS * i

        @pl.when(chunk < nchd)
        def _(chunk=chunk):
            pltpu.sync_copy(densh.at[pl.ds(chunk * 128, 128)],
                            den_out.at[pl.ds(c * kd + chunk * 128, 128)])


def _gat_edge(kpad, kd, buckets, comps, compd, cnt, als, ald, m, h):
    rows_t = kd // 16
    znum = jnp.zeros((rows_t, 128), _f32)
    zden = jnp.zeros((128,), _f32)
    kern = pl.kernel(
        functools.partial(_gat_edge_body, kpad, kd, buckets),
        out_type=(
            jax.ShapeDtypeStruct((NC * kd, 128), _f32),
            jax.ShapeDtypeStruct((NC * kd,), _f32),
        ),
        mesh=_sc_mesh(),
        compiler_params=pltpu.CompilerParams(needs_layout_passes=False),
        scratch_types=[
            pltpu.VMEM((kpad,), _f32), pltpu.VMEM((kpad,), _f32),
            pltpu.VMEM((kpad,), _f32),
            pltpu.VMEM((128,), _i32), pltpu.VMEM((128,), _i32),
            pltpu.VMEM((144,), _f32), pltpu.VMEM((128, 128), _f32),
            pltpu.VMEM((128,), _i32),
            pltpu.VMEM_SHARED((kpad, 128), _f32),
            pltpu.VMEM_SHARED((kd, 128), _f32),
            pltpu.VMEM_SHARED((kd,), _f32),
        ],
        name=f"gat_edge_{kpad}",
    )
    return kern(comps, compd, cnt, als, ald, m, h, znum, zden)


def _dec_agg_body(y_hbm, comps, compd, cnt_hbm, zrow_hbm, agg_out,
                  sbufw, dbufw, rowbuf, cntv, ysh, aggsh):
    c = lax.axis_index("c")
    s = lax.axis_index("s")
    wid = _wid()
    rows_y = KP3 // 16
    rows_t = KD3 // 16

    pltpu.sync_copy(cnt_hbm.at[pl.ds(wid * 128, 128)], cntv)
    pltpu.sync_copy(y_hbm.at[pl.ds(s * rows_y, rows_y)],
                    ysh.at[pl.ds(s * rows_y, rows_y)])
    pltpu.sync_copy(zrow_hbm, aggsh.at[pl.ds(s * rows_t, rows_t)])
    plsc.subcore_barrier()

    cb = cntv[pl.ds(0, 16)][0]
    segbase = wid * PTP

    def chunk_body(j, carry):
        @pl.when(j * 128 < cb)
        def _():
            pltpu.sync_copy(comps.at[pl.ds(segbase + j * 128, 128)], sbufw)
            pltpu.sync_copy(compd.at[pl.ds(segbase + j * 128, 128)], dbufw)
            for v in range(8):
                sv = sbufw[pl.ds(v * 16, 16)]
                dv = dbufw[pl.ds(v * 16, 16)]
                lanepos = j * 128 + v * 16 + lax.iota(_i32, 16)
                mvld = lanepos < cb
                sbufw[pl.ds(v * 16, 16)] = jnp.where(mvld, sv, 0)
                dbufw[pl.ds(v * 16, 16)] = jnp.where(mvld, dv, KP3)
            pltpu.sync_copy(ysh.at[sbufw], rowbuf)
            pltpu.sync_copy(rowbuf, aggsh.at[dbufw], add=True)

        return carry

    lax.fori_loop(0, NCH, chunk_body, 0)
    plsc.subcore_barrier()
    pltpu.sync_copy(aggsh.at[pl.ds(s * rows_t, rows_t)],
                    agg_out.at[pl.ds(c * KD3 + s * rows_t, rows_t)])


def _dec_agg(y, comps, compd, cnt):
    zrow = jnp.zeros((KD3 // 16, 128), _f32)
    kern = pl.kernel(
        _dec_agg_body,
        out_type=jax.ShapeDtypeStruct((NC * KD3, 128), _f32),
        mesh=_sc_mesh(),
        compiler_params=pltpu.CompilerParams(needs_layout_passes=False),
        scratch_types=[
            pltpu.VMEM((128,), _i32), pltpu.VMEM((128,), _i32),
            pltpu.VMEM((128, 128), _f32),
            pltpu.VMEM((128,), _i32),
            pltpu.VMEM_SHARED((KP3, 128), _f32),
            pltpu.VMEM_SHARED((KD3, 128), _f32),
        ],
        name="dec_agg",
    )
    return kern(y, comps, compd, cnt, zrow)


def _prep0_body(degT_ref, x4_ref, y_ref, dis_ref):
    degT = degT_ref[...]
    d = degT[:, 0:1] + degT[:, 1:2] + 1.0
    dis = lax.rsqrt(d)
    dis_ref[...] = dis
    y_ref[...] = dis * x4_ref[...]


def _prep0(degT, x4):
    return pl.pallas_call(
        _prep0_body,
        out_shape=(jax.ShapeDtypeStruct((NPAD0, 4), _f32),
                   jax.ShapeDtypeStruct((NPAD0, 1), _f32)),
    )(degT, x4)


def _gcn0_fin_body(agg_ref, x4_ref, dis_ref, W4_ref, b_ref, p_ref,
                   h_ref, score_ref):
    dis = dis_ref[...]
    agg = agg_ref[...]
    aggs = agg[0] + agg[1]
    pre = dis * aggs + (dis * dis) * x4_ref[...]
    h = jnp.dot(pre, W4_ref[...], preferred_element_type=_f32) + b_ref[...]
    h_ref[...] = h
    p = p_ref[...]
    pn = p * lax.rsqrt(jnp.sum(p * p))
    proj = jnp.dot(h, pn, preferred_element_type=_f32)
    rows = lax.broadcasted_iota(_i32, (NPAD0, 1), 0)
    score_ref[...] = jnp.where(rows < N0, jnp.tanh(proj), -1e30)


def _gcn0_fin(agg, x4, dis, W4, b, p):
    return pl.pallas_call(
        _gcn0_fin_body,
        out_shape=(jax.ShapeDtypeStruct((NPAD0, H), _f32),
                   jax.ShapeDtypeStruct((NPAD0, 1), _f32)),
    )(agg, x4, dis, W4, b, p)


def _rank_body(npad, scol_ref, srow_ref, out_ref):
    ib = pl.program_id(0)
    si = scol_ref[...]
    i_ids = ib * 512 + lax.broadcasted_iota(_i32, (512, 1), 0)

    def body(j, acc):
        sj = srow_ref[:, pl.ds(j * 512, 512)]
        j_ids = j * 512 + lax.broadcasted_iota(_i32, (1, 512), 1)
        gt = (sj > si).astype(_i32)
        eq = ((sj == si) & (j_ids < i_ids)).astype(_i32)
        return acc + jnp.sum(gt + eq, axis=1, keepdims=True)

    out_ref[...] = lax.fori_loop(0, npad // 512, body,
                                 jnp.zeros((512, 1), _i32))


def _rank(score_col, npad):
    srow = score_col.reshape(1, npad)
    return pl.pallas_call(
        functools.partial(_rank_body, npad),
        grid=(npad // 512,),
        in_specs=[pl.BlockSpec((512, 1), lambda i: (i, 0)),
                  pl.BlockSpec((1, npad), lambda i: (0, 0))],
        out_specs=pl.BlockSpec((512, 1), lambda i: (i, 0)),
        out_shape=jax.ShapeDtypeStruct((npad, 1), _i32),
    )(score_col, srow)


def _select_body(npad, k, rank_ref, srow_ref, h_ref, hsel_ref, vals_ref):
    rb = pl.program_id(0)
    r_ids = rb * 256 + lax.broadcasted_iota(_i32, (256, 1), 0)

    def body(ic, carry):
        acc, vacc = carry
        rk = rank_ref[:, pl.ds(ic * 512, 512)]
        sc = srow_ref[:, pl.ds(ic * 512, 512)]
        hit = (rk == r_ids) & (r_ids < k)
        P = jnp.where(hit, sc, 0.0)
        acc = acc + jnp.dot(P, h_ref[pl.ds(ic * 512, 512), :],
                            preferred_element_type=_f32)
        vacc = vacc + jnp.sum(P, axis=1, keepdims=True)
        return acc, vacc

    acc, vacc = lax.fori_loop(
        0, npad // 512, body,
        (jnp.zeros((256, H), _f32), jnp.zeros((256, 1), _f32)))
    hsel_ref[...] = acc
    vals_ref[...] = vacc


def _select(rank_col, score_col, h, npad, k, kpad):
    rrow = rank_col.reshape(1, npad)
    srow = score_col.reshape(1, npad)
    return pl.pallas_call(
        functools.partial(_select_body, npad, k),
        grid=(kpad // 256,),
        in_specs=[pl.BlockSpec((1, npad), lambda i: (0, 0)),
                  pl.BlockSpec((1, npad), lambda i: (0, 0)),
                  pl.BlockSpec((npad, H), lambda i: (0, 0))],
        out_specs=(pl.BlockSpec((256, H), lambda i: (i, 0)),
                   pl.BlockSpec((256, 1), lambda i: (i, 0))),
        out_shape=(jax.ShapeDtypeStruct((kpad, H), _f32),
                   jax.ShapeDtypeStruct((kpad, 1), _f32)),
    )(rrow, srow, h)


def _gat_dense_body(n, kpad, hsel_ref, W_ref, as_ref, ad_ref,
                    h_ref, als_ref, ald_ref, m_ref, ws_ref):
    rows = lax.broadcasted_iota(_i32, (kpad, 1), 0)
    h1 = hsel_ref[...]
    h = jnp.dot(h1, W_ref[...], preferred_element_type=_f32)
    h_ref[...] = h
    als = jnp.dot(h, as_ref[...], preferred_element_type=_f32)
    ald = jnp.dot(h, ad_ref[...], preferred_element_type=_f32)
    als_ref[...] = als
    ald_ref[...] = ald
    amax = jnp.max(jnp.where(rows < n, als, -3e38))
    pre_m = amax + ald
    m = jnp.where(pre_m > 0, pre_m, 0.2 * pre_m)
    m_ref[...] = m
    pre_s = als + ald
    es = jnp.where(pre_s > 0, pre_s, 0.2 * pre_s)
    ws_ref[...] = jnp.maximum(jnp.exp(es - m), 1e-30)


def _gat_dense(hsel, W, a_s, a_d, n, kpad):
    return pl.pallas_call(
        functools.partial(_gat_dense_body, n, kpad),
        out_shape=(jax.ShapeDtypeStruct((kpad, H), _f32),
                   jax.ShapeDtypeStruct((kpad, 1), _f32),
                   jax.ShapeDtypeStruct((kpad, 1), _f32),
                   jax.ShapeDtypeStruct((kpad, 1), _f32),
                   jax.ShapeDtypeStruct((kpad, 1), _f32)),
    )(hsel, W, a_s, a_d)


def _gat_fin_body(n, kpad, num_ref, den_ref, ws_ref, h_ref, b_ref, p_ref,
                  hout_ref, score_ref):
    ws = ws_ref[...]
    num_a = num_ref[...]
    den_a = den_ref[...]
    num = (num_a[0, :kpad, :] + num_a[1, :kpad, :] + ws * h_ref[...])
    den = (den_a[0, :kpad, :] + den_a[1, :kpad, :] + ws)
    out = num / den + b_ref[...]
    hout_ref[...] = out
    p = p_ref[...]
    pn = p * lax.rsqrt(jnp.sum(p * p))
    proj = jnp.dot(out, pn, preferred_element_type=_f32)
    rows = lax.broadcasted_iota(_i32, (kpad, 1), 0)
    score_ref[...] = jnp.where(rows < n, jnp.tanh(proj), -1e30)


def _gat_fin(num, den3, ws, h, b, p, n, kpad):
    return pl.pallas_call(
        functools.partial(_gat_fin_body, n, kpad),
        out_shape=(jax.ShapeDtypeStruct((kpad, H), _f32),
                   jax.ShapeDtypeStruct((kpad, 1), _f32)),
    )(num, den3, ws, h, b, p)


def _vae_body(degT_ref, hsel_ref, Wmu_ref, bmu_ref, Wlv_ref, blv_ref,
              Wld_ref, bld_ref, eps_ref,
              mu_ref, lv_ref, z_ref, y_ref, dis_ref):
    h5 = hsel_ref[...]
    mu = jnp.dot(h5, Wmu_ref[...], preferred_element_type=_f32) + bmu_ref[...]
    lv = jnp.dot(h5, Wlv_ref[...], preferred_element_type=_f32) + blv_ref[...]
    mu_ref[...] = mu
    lv_ref[...] = lv
    zlat = mu + eps_ref[...] * jnp.exp(0.5 * lv)
    z = jnp.dot(zlat, Wld_ref[...], preferred_element_type=_f32) + bld_ref[...]
    z_ref[...] = z
    degT = degT_ref[...]
    d = degT[:, 0:1] + degT[:, 1:2] + 2.0
    dis = lax.rsqrt(d)
    dis_ref[...] = dis
    y_ref[...] = dis * z


def _vae(degT, hsel, Wmu, bmu, Wlv, blv, Wld, bld, eps):
    return pl.pallas_call(
        _vae_body,
        out_shape=(jax.ShapeDtypeStruct((KP3, LAT), _f32),
                   jax.ShapeDtypeStruct((KP3, LAT), _f32),
                   jax.ShapeDtypeStruct((KP3, H), _f32),
                   jax.ShapeDtypeStruct((KP3, H), _f32),
                   jax.ShapeDtypeStruct((KP3, 1), _f32)),
    )(degT, hsel, Wmu, bmu, Wlv, blv, Wld, bld, eps)


def _dec_fin_body(agg_ref, z_ref, dis_ref, W_ref, b_ref, zn_ref, yn_ref):
    dis = dis_ref[...]
    agg = agg_ref[...]
    aggs = agg[0, :KP3, :] + agg[1, :KP3, :]
    pre = dis * aggs + 2.0 * (dis * dis) * z_ref[...]
    zn = jnp.dot(pre, W_ref[...], preferred_element_type=_f32) + b_ref[...]
    zn_ref[...] = zn
    yn_ref[...] = dis * zn


def _dec_fin(agg, z, dis, W, b):
    return pl.pallas_call(
        _dec_fin_body,
        out_shape=(jax.ShapeDtypeStruct((KP3, H), _f32),
                   jax.ShapeDtypeStruct((KP3, H), _f32)),
    )(agg, z, dis, W, b)


def kernel(x, edge_index, undirected_edge_index, batch, params):
    p = params
    src = edge_index[0]
    dst = edge_index[1]
    src2d = src.reshape(NSUB, 128)
    dst2d = dst.reshape(NSUB, 128)

    x4 = jnp.pad(x, ((0, NPAD0 - N0), (0, 1)))
    W4 = jnp.pad(p['W_e0'], ((0, 1), (0, 0)))

    deg, degdec, comps, compd, cnt = _edge_prep(src, dst)

    y0, dis0 = _prep0(deg.reshape(NC, NPAD0).T, x4)
    agg0 = _gcn0_agg(y0, src, dst).reshape(NC, NPAD0, 4)
    h0, score0 = _gcn0_fin(agg0, x4, dis0, W4, p['b_e0'].reshape(1, H),
                           p['p0'].reshape(H, 1))

    rk0 = _rank(score0, NPAD0)
    hsel1, _ = _select(rk0, score0, h0, NPAD0, K1, KP1)
    hg1, als1, ald1, m1, ws1 = _gat_dense(hsel1, p['W_e1'],
                                          p['a_src1'].reshape(H, 1),
                                          p['a_dst1'].reshape(H, 1), K1, KP1)
    num1, den1 = _gat_edge(KP1, KD1, (0, 1, 2), comps, compd, cnt,
                           als1.reshape(KP1), ald1.reshape(KP1),
                           m1.reshape(KP1), hg1)
    h1, score1 = _gat_fin(num1.reshape(NC, KD1, H),
                          den1.reshape(NC, KD1, 1), ws1, hg1,
                          p['b_e1'].reshape(1, H), p['p1'].reshape(H, 1),
                          K1, KP1)

    rk1 = _rank(score1, KP1)
    hsel2, _ = _select(rk1, score1, h1, KP1, K2, KP2)
    hg2, als2, ald2, m2, ws2 = _gat_dense(hsel2, p['W_e2'],
                                          p['a_src2'].reshape(H, 1),
                                          p['a_dst2'].reshape(H, 1), K2, KP2)
    num2, den2 = _gat_edge(KP2, KD2, (0, 1), comps, compd, cnt,
                           als2.reshape(KP2), ald2.reshape(KP2),
                           m2.reshape(KP2), hg2)
    h2, score2 = _gat_fin(num2.reshape(NC, KD2, H),
                          den2.reshape(NC, KD2, 1), ws2, hg2,
                          p['b_e2'].reshape(1, H), p['p2'].reshape(H, 1),
                          K2, KP2)

    rk2 = _rank(score2, KP2)
    hsel3, _ = _select(rk2, score2, h2, KP2, K3, KP3)
    eps = jax.random.normal(jax.random.key(42), (K3, LAT), _f32)
    eps = jnp.pad(eps, ((0, KP3 - K3), (0, 0)))
    degdecT = degdec.reshape(NC, KD3).T[:KP3, :]
    mu, lv, z, y, disd = _vae(degdecT, hsel3,
                              p['W_mu'], p['b_mu'].reshape(1, LAT),
                              p['W_lv'], p['b_lv'].reshape(1, LAT),
                              p['W_ld'], p['b_ld'].reshape(1, H), eps)

    for Wd, bd in [(p['W_d2'], p['b_d2']), (p['W_d1'], p['b_d1']),
                   (p['W_d0'], p['b_d0'])]:
        aggd = _dec_agg(y, comps, compd, cnt).reshape(NC, KD3, H)
        z, y = _dec_fin(aggd, z, disd, Wd, bd.reshape(1, H))

    return z[:K3], mu[:K3], lv[:K3]


# speedup vs baseline: 12.1240x; 12.1240x over previous
import functools

import jax
import jax.numpy as jnp
from jax import lax
from jax.experimental import pallas as pl
from jax.experimental.pallas import tpu as pltpu
from jax.experimental.pallas import tpu_sc as plsc

NC, NS = 2, 16
NW = NC * NS
N0, E, H, LAT = 10000, 640000, 128, 16
NPAD0 = 10240
PT = E // NW
ECH = 2560
NECH = E // ECH
PTP = 20608
NCH = PTP // 128
NSUB = E // 128
NBLK8 = E // (128 * 8)

K1, K2, K3 = 5000, 2500, 1250
KP1, KP2, KP3 = 5120, 2560, 1280
KD1, KD2, KD3 = 5248, 2688, 1408
CDUMP = 3 * NW * PTP

_f32 = jnp.float32
_i32 = jnp.int32


def _sc_mesh():
    return plsc.VectorSubcoreMesh(core_axis_name="c", subcore_axis_name="s")


def _wid():
    return lax.axis_index("s") * NC + lax.axis_index("c")


def _edge_prep_body(src_hbm, dst_hbm, one_hbm, z640_hbm,
                    deg_out, degdec_out, comps_out, compd_out, cnt_out,
                    sbuf, dbuf, pidx, vsb, vdb, si128, di128, w128, one128,
                    cntv, degsh, ddsh):
    c = lax.axis_index("c")
    s = lax.axis_index("s")
    wid = _wid()

    pltpu.sync_copy(one_hbm, one128)
    pltpu.sync_copy(z640_hbm, degsh.at[pl.ds(s * 640, 640)])

    @pl.when(s < KD3 // 128)
    def _():
        pltpu.sync_copy(z640_hbm.at[pl.ds(0, 128)], ddsh.at[pl.ds(s * 128, 128)])

    plsc.subcore_barrier()

    def deg_body(j, carry):
        sub = wid + NW * j

        @pl.when(sub < NSUB)
        def _():
            base = sub * 128
            pltpu.sync_copy(src_hbm.at[pl.ds(base, 128)], si128)
            pltpu.sync_copy(dst_hbm.at[pl.ds(base, 128)], di128)
            pltpu.sync_copy(one128, degsh.at[di128], add=True)
            for v in range(8):
                sv = si128[pl.ds(v * 16, 16)]
                dv = di128[pl.ds(v * 16, 16)]
                mA = jnp.maximum(sv, dv) < K3
                w128[pl.ds(v * 16, 16)] = jnp.where(mA, 1.0, 0.0).astype(_f32)
                di128[pl.ds(v * 16, 16)] = jnp.where(mA, dv, KP3)
            pltpu.sync_copy(w128, ddsh.at[di128], add=True)

        return carry

    lax.fori_loop(0, (NSUB + NW - 1) // NW, deg_body, 0)

    cuts = [(0, K3), (K3, K2), (K2, K1)]
    offs = []
    lanes16 = lax.iota(_i32, 16)
    for b, (lo, hi) in enumerate(cuts):
        seg = (b * NW + wid) * PTP

        def chunk_body(i, off, lo=lo, hi=hi, seg=seg):
            ch = wid + NW * i

            def do_chunk(off_in):
                pltpu.sync_copy(src_hbm.at[pl.ds(ch * ECH, ECH)], sbuf)
                pltpu.sync_copy(dst_hbm.at[pl.ds(ch * ECH, ECH)], dbuf)

                def grp_body(g, off2):
                    for v in range(8):
                        sv = sbuf[pl.ds(g * 128 + v * 16, 16)]
                        dv = dbuf[pl.ds(g * 128 + v * 16, 16)]
                        mx = jnp.maximum(sv, dv)
                        mb = (mx >= lo) & (mx < hi)
                        cs = plsc.cumsum(mb.astype(_i32))
                        pos = jnp.where(mb, seg + off2 + cs - 1,
                                        CDUMP + wid * 128 + v * 16 + lanes16)
                        pidx[pl.ds(v * 16, 16)] = pos
                        vsb[pl.ds(v * 16, 16)] = sv
                        vdb[pl.ds(v * 16, 16)] = dv
                        off2 = off2 + cs[15]
                    pltpu.sync_copy(vsb, comps_out.at[pidx])
                    pltpu.sync_copy(vdb, compd_out.at[pidx])
                    return off2

                return lax.fori_loop(0, ECH // 128, grp_body, off_in)

            return lax.cond(ch < NECH, do_chunk, lambda o: o, off)

        off_b = lax.fori_loop(0, (NECH + NW - 1) // NW, chunk_body,
                              jnp.int32(0))
        offs.append(off_b)

    lanes = lax.iota(_i32, 16)
    cvec = jnp.zeros((16,), _i32)
    for b, off_b in enumerate(offs):
        cvec = jnp.where(lanes == b, jnp.zeros((16,), _i32) + off_b, cvec)
    cntv[pl.ds(0, 16)] = cvec
    for t in range(1, 8):
        cntv[pl.ds(t * 16, 16)] = jnp.zeros((16,), _i32)
    pltpu.sync_copy(cntv, cnt_out.at[pl.ds(wid * 128, 128)])

    plsc.subcore_barrier()

    @pl.when(s == 0)
    def _():
        pltpu.sync_copy(degsh, deg_out.at[pl.ds(c * NPAD0, NPAD0)])
        pltpu.sync_copy(ddsh, degdec_out.at[pl.ds(c * KD3, KD3)])


def _edge_prep(src, dst):
    ones = jnp.ones((128,), _f32)
    z640 = jnp.zeros((640,), _f32)
    kern = pl.kernel(
        _edge_prep_body,
        out_type=(
            jax.ShapeDtypeStruct((NC * NPAD0,), _f32),
            jax.ShapeDtypeStruct((NC * KD3,), _f32),
            jax.ShapeDtypeStruct((CDUMP + NW * 128,), _i32),
            jax.ShapeDtypeStruct((CDUMP + NW * 128,), _i32),
            jax.ShapeDtypeStruct((NW * 128,), _i32),
        ),
        mesh=_sc_mesh(),
        compiler_params=pltpu.CompilerParams(needs_layout_passes=False),
        scratch_types=[
            pltpu.VMEM((ECH,), _i32), pltpu.VMEM((ECH,), _i32),
            pltpu.VMEM((128,), _i32), pltpu.VMEM((128,), _i32),
            pltpu.VMEM((128,), _i32),
            pltpu.VMEM((128,), _i32), pltpu.VMEM((128,), _i32),
            pltpu.VMEM((128,), _f32), pltpu.VMEM((128,), _f32),
            pltpu.VMEM((128,), _i32),
            pltpu.VMEM_SHARED((NPAD0,), _f32),
            pltpu.VMEM_SHARED((KD3,), _f32),
        ],
        name="edge_prep",
    )
    return kern(src, dst, ones, z640)


def _gcn0_agg_body(y_hbm, src_hbm, dst_hbm, z640_hbm, agg_out,
                   si128, di128, gidx, sidx, gbuf, aggsh):
    c = lax.axis_index("c")
    s = lax.axis_index("s")
    wid = _wid()

    for q in range(4):
        pltpu.sync_copy(z640_hbm, aggsh.at[pl.ds(s * 2560 + q * 640, 640)])
    plsc.subcore_barrier()

    def blk_body(j, carry):
        sub = wid + NW * j

        @pl.when(sub < NSUB)
        def _():
            base = sub * 128
            pltpu.sync_copy(src_hbm.at[pl.ds(base, 128)], si128)
            pltpu.sync_copy(dst_hbm.at[pl.ds(base, 128)], di128)
            for cc in range(4):
                for v in range(8):
                    sv = si128[pl.ds(v * 16, 16)]
                    dv = di128[pl.ds(v * 16, 16)]
                    gidx[pl.ds(v * 16, 16)] = sv * 4 + cc
                    sidx[pl.ds(v * 16, 16)] = dv * 4 + cc
                pltpu.sync_copy(y_hbm.at[gidx], gbuf)
                pltpu.sync_copy(gbuf, aggsh.at[sidx], add=True)

        return carry

    lax.fori_loop(0, (NSUB + NW - 1) // NW, blk_body, 0)
    plsc.subcore_barrier()
    pltpu.sync_copy(aggsh.at[pl.ds(s * 2560, 2560)],
                    agg_out.at[pl.ds(c * NPAD0 * 4 + s * 2560, 2560)])


def _gcn0_agg(y, src, dst):
    z640 = jnp.zeros((640,), _f32)
    kern = pl.kernel(
        _gcn0_agg_body,
        out_type=jax.ShapeDtypeStruct((NC * NPAD0 * 4,), _f32),
        mesh=_sc_mesh(),
        compiler_params=pltpu.CompilerParams(needs_layout_passes=False),
        scratch_types=[
            pltpu.VMEM((128,), _i32), pltpu.VMEM((128,), _i32),
            pltpu.VMEM((128,), _i32), pltpu.VMEM((128,), _i32),
            pltpu.VMEM((128,), _f32),
            pltpu.VMEM_SHARED((NPAD0 * 4,), _f32),
        ],
        name="gcn0_agg",
    )
    return kern(y.reshape(NPAD0 * 4), src, dst, z640)


def _gat_edge_body(kpad, kd, buckets,
                   comps, compd, cnt_hbm, als_hbm, ald_hbm, m_hbm, h_hbm,
                   znum_hbm, zden_hbm,
                   num_out, den_out,
                   alsv, aldv, mv, sbufw, dbufw, wbuf, rowbuf,
                   cntv, hsh, numsh, densh):
    c = lax.axis_index("c")
    s = lax.axis_index("s")
    wid = _wid()
    rows_h = kpad // 16
    rows_t = kd // 16
    dump = kpad

    nchd = kd // 128
    pltpu.sync_copy(als_hbm, alsv)
    pltpu.sync_copy(ald_hbm, aldv)
    pltpu.sync_copy(m_hbm, mv)
    pltpu.sync_copy(cnt_hbm.at[pl.ds(wid * 128, 128)], cntv)
    pltpu.sync_copy(h_hbm.at[pl.ds(s * rows_h, rows_h)],
                    hsh.at[pl.ds(s * rows_h, rows_h)])
    pltpu.sync_copy(znum_hbm, numsh.at[pl.ds(s * rows_t, rows_t)])
    for i in range((nchd + NS - 1) // NS):
        chunk = s + NS * i

        @pl.when(chunk < nchd)
        def _(chunk=chunk):
            pltpu.sync_copy(zden_hbm, densh.at[pl.ds(chunk * 128, 128)])

    plsc.subcore_barrier()

    cntvec = cntv[pl.ds(0, 16)]
    for b in buckets:
        cb = cntvec[b]
        segbase = (b * NW + wid) * PTP

        def chunk_body(j, carry, cb=cb, segbase=segbase):
            @pl.when(j * 128 < cb)
            def _():
                pltpu.sync_copy(comps.at[pl.ds(segbase + j * 128, 128)], sbufw)
                pltpu.sync_copy(compd.at[pl.ds(segbase + j * 128, 128)], dbufw)
                for v in range(8):
                    sv = sbufw[pl.ds(v * 16, 16)]
                    dv = dbufw[pl.ds(v * 16, 16)]
                    lanepos = j * 128 + v * 16 + lax.iota(_i32, 16)
                    mvld = lanepos < cb
                    s_c = jnp.where(mvld, sv, 0)
                    d_t = jnp.where(mvld, dv, 0)
                    a1 = plsc.load_gather(alsv, [s_c])
                    a2 = plsc.load_gather(aldv, [d_t])
                    mm = plsc.load_gather(mv, [d_t])
                    e = a1 + a2
                    e = jnp.where(e > 0, e, 0.2 * e)
                    w = jnp.where(mvld, jnp.exp(e - mm), 0.0)
                    wbuf[pl.ds(v * 16, 16)] = w
                    sbufw[pl.ds(v * 16, 16)] = s_c
                    dbufw[pl.ds(v * 16, 16)] = jnp.where(mvld, dv, dump)
                pltpu.sync_copy(wbuf.at[pl.ds(0, 128)], densh.at[dbufw],
                                add=True)
                pltpu.sync_copy(hsh.at[sbufw], rowbuf)

                def scale_row(r, carry2):
                    wr = wbuf[pl.ds(r, 16)][0]
                    for cc in range(8):
                        rowbuf[r, pl.ds(cc * 16, 16)] = (
                            rowbuf[r, pl.ds(cc * 16, 16)] * wr)
                    return carry2

                lax.fori_loop(0, 128, scale_row, 0)
                pltpu.sync_copy(rowbuf, numsh.at[dbufw], add=True)

            return carry

        lax.fori_loop(0, NCH, chunk_body, 0)

    plsc.subcore_barrier()
    pltpu.sync_copy(numsh.at[pl.ds(s * rows_t, rows_t)],
                    num_out.at[pl.ds(c * kd + s * rows_t, rows_t)])
    for i in range((nchd + NS - 1) // NS):
        chunk = s + NS * i

        @pl.when(chunk < nchd)
        def _(chunk=chunk):
            pltpu.sync_copy(densh.at[pl.ds(chunk * 128, 128)],
                            den_out.at[pl.ds(c * kd + chunk * 128, 128)])


def _gat_edge(kpad, kd, buckets, comps, compd, cnt, als, ald, m, h):
    rows_t = kd // 16
    znum = jnp.zeros((rows_t, 128), _f32)
    zden = jnp.zeros((128,), _f32)
    kern = pl.kernel(
        functools.partial(_gat_edge_body, kpad, kd, buckets),
        out_type=(
            jax.ShapeDtypeStruct((NC * kd, 128), _f32),
            jax.ShapeDtypeStruct((NC * kd,), _f32),
        ),
        mesh=_sc_mesh(),
        compiler_params=pltpu.CompilerParams(needs_layout_passes=False),
        scratch_types=[
            pltpu.VMEM((kpad,), _f32), pltpu.VMEM((kpad,), _f32),
            pltpu.VMEM((kpad,), _f32),
            pltpu.VMEM((128,), _i32), pltpu.VMEM((128,), _i32),
            pltpu.VMEM((144,), _f32), pltpu.VMEM((128, 128), _f32),
            pltpu.VMEM((128,), _i32),
            pltpu.VMEM_SHARED((kpad, 128), _f32),
            pltpu.VMEM_SHARED((kd, 128), _f32),
            pltpu.VMEM_SHARED((kd,), _f32),
        ],
        name=f"gat_edge_{kpad}",
    )
    return kern(comps, compd, cnt, als, ald, m, h, znum, zden)


def _dec_agg_body(y_hbm, comps, compd, cnt_hbm, zrow_hbm, agg_out,
                  sbufw, dbufw, rowbuf, cntv, ysh, aggsh):
    c = lax.axis_index("c")
    s = lax.axis_index("s")
    wid = _wid()
    rows_y = KP3 // 16
    rows_t = KD3 // 16

    pltpu.sync_copy(cnt_hbm.at[pl.ds(wid * 128, 128)], cntv)
    pltpu.sync_copy(y_hbm.at[pl.ds(s * rows_y, rows_y)],
                    ysh.at[pl.ds(s * rows_y, rows_y)])
    pltpu.sync_copy(zrow_hbm, aggsh.at[pl.ds(s * rows_t, rows_t)])
    plsc.subcore_barrier()

    cb = cntv[pl.ds(0, 16)][0]
    segbase = wid * PTP

    def chunk_body(j, carry):
        @pl.when(j * 128 < cb)
        def _():
            pltpu.sync_copy(comps.at[pl.ds(segbase + j * 128, 128)], sbufw)
            pltpu.sync_copy(compd.at[pl.ds(segbase + j * 128, 128)], dbufw)
            for v in range(8):
                sv = sbufw[pl.ds(v * 16, 16)]
                dv = dbufw[pl.ds(v * 16, 16)]
                lanepos = j * 128 + v * 16 + lax.iota(_i32, 16)
                mvld = lanepos < cb
                sbufw[pl.ds(v * 16, 16)] = jnp.where(mvld, sv, 0)
                dbufw[pl.ds(v * 16, 16)] = jnp.where(mvld, dv, KP3)
            pltpu.sync_copy(ysh.at[sbufw], rowbuf)
            pltpu.sync_copy(rowbuf, aggsh.at[dbufw], add=True)

        return carry

    lax.fori_loop(0, NCH, chunk_body, 0)
    plsc.subcore_barrier()
    pltpu.sync_copy(aggsh.at[pl.ds(s * rows_t, rows_t)],
                    agg_out.at[pl.ds(c * KD3 + s * rows_t, rows_t)])


def _dec_agg(y, comps, compd, cnt):
    zrow = jnp.zeros((KD3 // 16, 128), _f32)
    kern = pl.kernel(
        _dec_agg_body,
        out_type=jax.ShapeDtypeStruct((NC * KD3, 128), _f32),
        mesh=_sc_mesh(),
        compiler_params=pltpu.CompilerParams(needs_layout_passes=False),
        scratch_types=[
            pltpu.VMEM((128,), _i32), pltpu.VMEM((128,), _i32),
            pltpu.VMEM((128, 128), _f32),
            pltpu.VMEM((128,), _i32),
            pltpu.VMEM_SHARED((KP3, 128), _f32),
            pltpu.VMEM_SHARED((KD3, 128), _f32),
        ],
        name="dec_agg",
    )
    return kern(y, comps, compd, cnt, zrow)


def _prep0_body(degT_ref, x4_ref, y_ref, dis_ref):
    degT = degT_ref[...]
    d = degT[:, 0:1] + degT[:, 1:2] + 1.0
    dis = lax.rsqrt(d)
    dis_ref[...] = dis
    y_ref[...] = dis * x4_ref[...]


def _prep0(degT, x4):
    return pl.pallas_call(
        _prep0_body,
        out_shape=(jax.ShapeDtypeStruct((NPAD0, 4), _f32),
                   jax.ShapeDtypeStruct((NPAD0, 1), _f32)),
    )(degT, x4)


def _gcn0_fin_body(agg_ref, x4_ref, dis_ref, W4_ref, b_ref, p_ref,
                   h_ref, score_ref):
    dis = dis_ref[...]
    agg = agg_ref[...]
    aggs = agg[0] + agg[1]
    pre = dis * aggs + (dis * dis) * x4_ref[...]
    h = jnp.dot(pre, W4_ref[...], preferred_element_type=_f32) + b_ref[...]
    h_ref[...] = h
    p = p_ref[...]
    pn = p * lax.rsqrt(jnp.sum(p * p))
    proj = jnp.dot(h, pn, preferred_element_type=_f32)
    rows = lax.broadcasted_iota(_i32, (NPAD0, 1), 0)
    score_ref[...] = jnp.where(rows < N0, jnp.tanh(proj), -1e30)


def _gcn0_fin(agg, x4, dis, W4, b, p):
    return pl.pallas_call(
        _gcn0_fin_body,
        out_shape=(jax.ShapeDtypeStruct((NPAD0, H), _f32),
                   jax.ShapeDtypeStruct((NPAD0, 1), _f32)),
    )(agg, x4, dis, W4, b, p)


def _rank_body(npad, scol_ref, srow_ref, out_ref):
    ib = pl.program_id(0)
    si = scol_ref[...]
    i_ids = ib * 512 + lax.broadcasted_iota(_i32, (512, 1), 0)

    def body(j, acc):
        sj = srow_ref[:, pl.ds(j * 512, 512)]
        j_ids = j * 512 + lax.broadcasted_iota(_i32, (1, 512), 1)
        gt = (sj > si).astype(_i32)
        eq = ((sj == si) & (j_ids < i_ids)).astype(_i32)
        return acc + jnp.sum(gt + eq, axis=1, keepdims=True)

    out_ref[...] = lax.fori_loop(0, npad // 512, body,
                                 jnp.zeros((512, 1), _i32))


def _rank(score_col, npad):
    srow = score_col.reshape(1, npad)
    return pl.pallas_call(
        functools.partial(_rank_body, npad),
        grid=(npad // 512,),
        in_specs=[pl.BlockSpec((512, 1), lambda i: (i, 0)),
                  pl.BlockSpec((1, npad), lambda i: (0, 0))],
        out_specs=pl.BlockSpec((512, 1), lambda i: (i, 0)),
        out_shape=jax.ShapeDtypeStruct((npad, 1), _i32),
    )(score_col, srow)


def _select_body(npad, k, rank_ref, srow_ref, h_ref, hsel_ref, vals_ref):
    rb = pl.program_id(0)
    r_ids = rb * 256 + lax.broadcasted_iota(_i32, (256, 1), 0)

    def body(ic, carry):
        acc, vacc = carry
        rk = rank_ref[:, pl.ds(ic * 512, 512)]
        sc = srow_ref[:, pl.ds(ic * 512, 512)]
        hit = (rk == r_ids) & (r_ids < k)
        P = jnp.where(hit, sc, 0.0)
        acc = acc + jnp.dot(P, h_ref[pl.ds(ic * 512, 512), :],
                            preferred_element_type=_f32)
        vacc = vacc + jnp.sum(P, axis=1, keepdims=True)
        return acc, vacc

    acc, vacc = lax.fori_loop(
        0, npad // 512, body,
        (jnp.zeros((256, H), _f32), jnp.zeros((256, 1), _f32)))
    hsel_ref[...] = acc
    vals_ref[...] = vacc


def _select(rank_col, score_col, h, npad, k, kpad):
    rrow = rank_col.reshape(1, npad)
    srow = score_col.reshape(1, npad)
    return pl.pallas_call(
        functools.partial(_select_body, npad, k),
        grid=(kpad // 256,),
        in_specs=[pl.BlockSpec((1, npad), lambda i: (0, 0)),
                  pl.BlockSpec((1, npad), lambda i: (0, 0)),
                  pl.BlockSpec((npad, H), lambda i: (0, 0))],
        out_specs=(pl.BlockSpec((256, H), lambda i: (i, 0)),
                   pl.BlockSpec((256, 1), lambda i: (i, 0))),
        out_shape=(jax.ShapeDtypeStruct((kpad, H), _f32),
                   jax.ShapeDtypeStruct((kpad, 1), _f32)),
    )(rrow, srow, h)


def _gat_dense_body(n, kpad, hsel_ref, W_ref, as_ref, ad_ref,
                    h_ref, als_ref, ald_ref, m_ref, ws_ref):
    rows = lax.broadcasted_iota(_i32, (kpad, 1), 0)
    h1 = hsel_ref[...]
    h = jnp.dot(h1, W_ref[...], preferred_element_type=_f32)
    h_ref[...] = h
    als = jnp.dot(h, as_ref[...], preferred_element_type=_f32)
    ald = jnp.dot(h, ad_ref[...], preferred_element_type=_f32)
    als_ref[...] = als
    ald_ref[...] = ald
    amax = jnp.max(jnp.where(rows < n, als, -3e38))
    pre_m = amax + ald
    m = jnp.where(pre_m > 0, pre_m, 0.2 * pre_m)
    m_ref[...] = m
    pre_s = als + ald
    es = jnp.where(pre_s > 0, pre_s, 0.2 * pre_s)
    ws_ref[...] = jnp.maximum(jnp.exp(es - m), 1e-30)


def _gat_dense(hsel, W, a_s, a_d, n, kpad):
    return pl.pallas_call(
        functools.partial(_gat_dense_body, n, kpad),
        out_shape=(jax.ShapeDtypeStruct((kpad, H), _f32),
                   jax.ShapeDtypeStruct((kpad, 1), _f32),
                   jax.ShapeDtypeStruct((kpad, 1), _f32),
                   jax.ShapeDtypeStruct((kpad, 1), _f32),
                   jax.ShapeDtypeStruct((kpad, 1), _f32)),
    )(hsel, W, a_s, a_d)


def _gat_fin_body(n, kpad, num_ref, den_ref, ws_ref, h_ref, b_ref, p_ref,
                  hout_ref, score_ref):
    ws = ws_ref[...]
    num_a = num_ref[...]
    den_a = den_ref[...]
    num = (num_a[0, :kpad, :] + num_a[1, :kpad, :] + ws * h_ref[...])
    den = (den_a[0, :kpad, :] + den_a[1, :kpad, :] + ws)
    out = num / den + b_ref[...]
    hout_ref[...] = out
    p = p_ref[...]
    pn = p * lax.rsqrt(jnp.sum(p * p))
    proj = jnp.dot(out, pn, preferred_element_type=_f32)
    rows = lax.broadcasted_iota(_i32, (kpad, 1), 0)
    score_ref[...] = jnp.where(rows < n, jnp.tanh(proj), -1e30)


def _gat_fin(num, den3, ws, h, b, p, n, kpad):
    return pl.pallas_call(
        functools.partial(_gat_fin_body, n, kpad),
        out_shape=(jax.ShapeDtypeStruct((kpad, H), _f32),
                   jax.ShapeDtypeStruct((kpad, 1), _f32)),
    )(num, den3, ws, h, b, p)


def _vae_body(degT_ref, hsel_ref, Wmu_ref, bmu_ref, Wlv_ref, blv_ref,
              Wld_ref, bld_ref, eps_ref,
              mu_ref, lv_ref, z_ref, y_ref, dis_ref):
    h5 = hsel_ref[...]
    mu = jnp.dot(h5, Wmu_ref[...], preferred_element_type=_f32) + bmu_ref[...]
    lv = jnp.dot(h5, Wlv_ref[...], preferred_element_type=_f32) + blv_ref[...]
    mu_ref[...] = mu
    lv_ref[...] = lv
    zlat = mu + eps_ref[...] * jnp.exp(0.5 * lv)
    z = jnp.dot(zlat, Wld_ref[...], preferred_element_type=_f32) + bld_ref[...]
    z_ref[...] = z
    degT = degT_ref[...]
    d = degT[:, 0:1] + degT[:, 1:2] + 2.0
    dis = lax.rsqrt(d)
    dis_ref[...] = dis
    y_ref[...] = dis * z


def _vae(degT, hsel, Wmu, bmu, Wlv, blv, Wld, bld, eps):
    return pl.pallas_call(
        _vae_body,
        out_shape=(jax.ShapeDtypeStruct((KP3, LAT), _f32),
                   jax.ShapeDtypeStruct((KP3, LAT), _f32),
                   jax.ShapeDtypeStruct((KP3, H), _f32),
                   jax.ShapeDtypeStruct((KP3, H), _f32),
                   jax.ShapeDtypeStruct((KP3, 1), _f32)),
    )(degT, hsel, Wmu, bmu, Wlv, blv, Wld, bld, eps)


def _dec_fin_body(agg_ref, z_ref, dis_ref, W_ref, b_ref, zn_ref, yn_ref):
    dis = dis_ref[...]
    agg = agg_ref[...]
    aggs = agg[0, :KP3, :] + agg[1, :KP3, :]
    pre = dis * aggs + 2.0 * (dis * dis) * z_ref[...]
    zn = jnp.dot(pre, W_ref[...], preferred_element_type=_f32) + b_ref[...]
    zn_ref[...] = zn
    yn_ref[...] = dis * zn


def _dec_fin(agg, z, dis, W, b):
    return pl.pallas_call(
        _dec_fin_body,
        out_shape=(jax.ShapeDtypeStruct((KP3, H), _f32),
                   jax.ShapeDtypeStruct((KP3, H), _f32)),
    )(agg, z, dis, W, b)


def kernel(x, edge_index, undirected_edge_index, batch, params):
    p = params
    src = edge_index[0]
    dst = edge_index[1]
    src2d = src.reshape(NSUB, 128)
    dst2d = dst.reshape(NSUB, 128)

    x4 = jnp.pad(x, ((0, NPAD0 - N0), (0, 1)))
    W4 = jnp.pad(p['W_e0'], ((0, 1), (0, 0)))

    deg, degdec, comps, compd, cnt = _edge_prep(src, dst)

    y0, dis0 = _prep0(deg.reshape(NC, NPAD0).T, x4)
    agg0 = _gcn0_agg(y0, src, dst).reshape(NC, NPAD0, 4)
    h0, score0 = _gcn0_fin(agg0, x4, dis0, W4, p['b_e0'].reshape(1, H),
                           p['p0'].reshape(H, 1))

    rk0 = _rank(score0, NPAD0)
    hsel1, _ = _select(rk0, score0, h0, NPAD0, K1, KP1)
    hg1, als1, ald1, m1, ws1 = _gat_dense(hsel1, p['W_e1'],
                                          p['a_src1'].reshape(H, 1),
                                          p['a_dst1'].reshape(H, 1), K1, KP1)
    num1, den1 = _gat_edge(KP1, KD1, (0, 1, 2), comps, compd, cnt,
                           als1.reshape(KP1), ald1.reshape(KP1),
                           m1.reshape(KP1), hg1)
    h1, score1 = _gat_fin(num1.reshape(NC, KD1, H),
                          den1.reshape(NC, KD1, 1), ws1, hg1,
                          p['b_e1'].reshape(1, H), p['p1'].reshape(H, 1),
                          K1, KP1)

    rk1 = _rank(score1, KP1)
    hsel2, _ = _select(rk1, score1, h1, KP1, K2, KP2)
    hg2, als2, ald2, m2, ws2 = _gat_dense(hsel2, p['W_e2'],
                                          p['a_src2'].reshape(H, 1),
                                          p['a_dst2'].reshape(H, 1), K2, KP2)
    num2, den2 = _gat_edge(KP2, KD2, (0, 1), comps, compd, cnt,
                           als2.reshape(KP2), ald2.reshape(KP2),
                           m2.reshape(KP2), hg2)
    h2, score2 = _gat_fin(num2.reshape(NC, KD2, H),
                          den2.reshape(NC, KD2, 1), ws2, hg2,
                          p['b_e2'].reshape(1, H), p['p2'].reshape(H, 1),
                          K2, KP2)

    rk2 = _rank(score2, KP2)
    hsel3, _ = _select(rk2, score2, h2, KP2, K3, KP3)
    eps = jax.random.normal(jax.random.key(42), (K3, LAT), _f32)
    eps = jnp.pad(eps, ((0, KP3 - K3), (0, 0)))
    degdecT = degdec.reshape(NC, KD3).T[:KP3, :]
    mu, lv, z, y, disd = _vae(degdecT, hsel3,
                              p['W_mu'], p['b_mu'].reshape(1, LAT),
                              p['W_lv'], p['b_lv'].reshape(1, LAT),
                              p['W_ld'], p['b_ld'].reshape(1, H), eps)

    for Wd, bd in [(p['W_d2'], p['b_d2']), (p['W_d1'], p['b_d1']),
                   (p['W_d0'], p['b_d0'])]:
        aggd = _dec_agg(y, comps, compd, cnt).reshape(NC, KD3, H)
        z, y = _dec_fin(aggd, z, disd, Wd, bd.reshape(1, H))

    return z[:K3], mu[:K3], lv[:K3]


# speedup vs baseline: 137.2505x; 11.3206x over previous
import functools

import jax
import jax.numpy as jnp
from jax import lax
from jax.experimental import pallas as pl
from jax.experimental.pallas import tpu as pltpu
from jax.experimental.pallas import tpu_sc as plsc

NC, NS = 2, 16
NW = NC * NS
N0, E, H, LAT = 10000, 640000, 128, 16
NPAD0 = 10240
PT = E // NW
ECH = 2560
NECH = E // ECH
PTP = 20608
NCH = PTP // 128
NSUB = E // 128
NBLK8 = E // (128 * 8)

K1, K2, K3 = 5000, 2500, 1250
KP1, KP2, KP3 = 5120, 2560, 1280
KD1, KD2, KD3 = 5248, 2688, 1408
CDUMP = 3 * NW * PTP

_f32 = jnp.float32
_i32 = jnp.int32


def _sc_mesh():
    return plsc.VectorSubcoreMesh(core_axis_name="c", subcore_axis_name="s")


def _wid():
    return lax.axis_index("s") * NC + lax.axis_index("c")


_SPD = NS * 3 * 2 * PTP


def _edge_prep_body(src_hbm, dst_hbm, one_hbm, z640_hbm,
                    deg_out, degdec_out, comps_out, compd_out, cnt_out,
                    sbuf, dbuf, pidx, pdidx, degidx, ddidx, ddval, one128,
                    cntv, degsh, ddsh, spcomp):
    c = lax.axis_index("c")
    s = lax.axis_index("s")
    wid = _wid()

    pltpu.sync_copy(one_hbm, one128)
    pltpu.sync_copy(z640_hbm, degsh.at[pl.ds(s * 640, 640)])

    @pl.when(s < KD3 // 128)
    def _():
        pltpu.sync_copy(z640_hbm.at[pl.ds(0, 128)], ddsh.at[pl.ds(s * 128, 128)])

    plsc.subcore_barrier()

    lanes16 = lax.iota(_i32, 16)
    baseA = (s * 3 + 0) * 2 * PTP
    baseB = (s * 3 + 1) * 2 * PTP
    baseC = (s * 3 + 2) * 2 * PTP
    dumpbase = _SPD + s * 128

    def chunk_body(i, offs):
        ch = wid + NW * i

        def do_chunk(offs_in):
            pltpu.sync_copy(src_hbm.at[pl.ds(ch * ECH, ECH)], sbuf)
            pltpu.sync_copy(dst_hbm.at[pl.ds(ch * ECH, ECH)], dbuf)

            def grp_body(g, offs2):
                offA, offB, offC = offs2
                for v in range(8):
                    sv = sbuf[pl.ds(g * 128 + v * 16, 16)]
                    dv = dbuf[pl.ds(g * 128 + v * 16, 16)]
                    mx = jnp.maximum(sv, dv)
                    mA = mx < K3
                    mB = (mx >= K3) & (mx < K2)
                    mC = (mx >= K2) & (mx < K1)
                    valid = mx < K1
                    csA = plsc.cumsum(mA.astype(_i32))
                    csB = plsc.cumsum(mB.astype(_i32))
                    csC = plsc.cumsum(mC.astype(_i32))
                    dumpv = dumpbase + v * 16 + lanes16
                    posS = jnp.where(
                        mA, baseA + offA + csA - 1,
                        jnp.where(mB, baseB + offB + csB - 1,
                                  jnp.where(mC, baseC + offC + csC - 1,
                                            dumpv)))
                    posD = posS + jnp.where(valid, PTP, 0)
                    pidx[pl.ds(v * 16, 16)] = posS
                    pdidx[pl.ds(v * 16, 16)] = posD
                    degidx[pl.ds(v * 16, 16)] = dv
                    ddidx[pl.ds(v * 16, 16)] = jnp.where(mA, dv, KP3)
                    ddval[pl.ds(v * 16, 16)] = jnp.where(mA, 1.0, 0.0).astype(_f32)
                    offA = offA + csA[15]
                    offB = offB + csB[15]
                    offC = offC + csC[15]
                vals_s = sbuf.at[pl.ds(g * 128, 128)]
                vals_d = dbuf.at[pl.ds(g * 128, 128)]
                pltpu.sync_copy(vals_s, spcomp.at[pidx])
                pltpu.sync_copy(vals_d, spcomp.at[pdidx])
                pltpu.sync_copy(one128, degsh.at[degidx], add=True)
                pltpu.sync_copy(ddval, ddsh.at[ddidx], add=True)
                return offA, offB, offC

            return lax.fori_loop(0, ECH // 128, grp_body, offs_in)

        return lax.cond(ch < NECH, do_chunk, lambda o: o, offs)

    z = jnp.int32(0)
    offA, offB, offC = lax.fori_loop(0, (NECH + NW - 1) // NW, chunk_body,
                                     (z, z, z))

    for b, base in enumerate((baseA, baseB, baseC)):
        seg = (b * NW + wid) * PTP
        pltpu.sync_copy(spcomp.at[pl.ds(base, PTP)],
                        comps_out.at[pl.ds(seg, PTP)])
        pltpu.sync_copy(spcomp.at[pl.ds(base + PTP, PTP)],
                        compd_out.at[pl.ds(seg, PTP)])

    lanes = lax.iota(_i32, 16)
    cvec = jnp.zeros((16,), _i32)
    for b, off_b in enumerate((offA, offB, offC)):
        cvec = jnp.where(lanes == b, jnp.zeros((16,), _i32) + off_b, cvec)
    cntv[pl.ds(0, 16)] = cvec
    for t in range(1, 8):
        cntv[pl.ds(t * 16, 16)] = jnp.zeros((16,), _i32)
    pltpu.sync_copy(cntv, cnt_out.at[pl.ds(wid * 128, 128)])

    plsc.subcore_barrier()

    @pl.when(s == 0)
    def _():
        pltpu.sync_copy(degsh, deg_out.at[pl.ds(c * NPAD0, NPAD0)])
        pltpu.sync_copy(ddsh, degdec_out.at[pl.ds(c * KD3, KD3)])


def _edge_prep(src, dst):
    ones = jnp.ones((128,), _f32)
    z640 = jnp.zeros((640,), _f32)
    kern = pl.kernel(
        _edge_prep_body,
        out_type=(
            jax.ShapeDtypeStruct((NC * NPAD0,), _f32),
            jax.ShapeDtypeStruct((NC * KD3,), _f32),
            jax.ShapeDtypeStruct((CDUMP,), _i32),
            jax.ShapeDtypeStruct((CDUMP,), _i32),
            jax.ShapeDtypeStruct((NW * 128,), _i32),
        ),
        mesh=_sc_mesh(),
        compiler_params=pltpu.CompilerParams(needs_layout_passes=False),
        scratch_types=[
            pltpu.VMEM((ECH,), _i32), pltpu.VMEM((ECH,), _i32),
            pltpu.VMEM((128,), _i32), pltpu.VMEM((128,), _i32),
            pltpu.VMEM((128,), _i32), pltpu.VMEM((128,), _i32),
            pltpu.VMEM((128,), _f32), pltpu.VMEM((128,), _f32),
            pltpu.VMEM((128,), _i32),
            pltpu.VMEM_SHARED((NPAD0,), _f32),
            pltpu.VMEM_SHARED((KD3,), _f32),
            pltpu.VMEM_SHARED((_SPD + NS * 128,), _i32),
        ],
        name="edge_prep",
    )
    return kern(src, dst, ones, z640)


def _gcn0_agg_body(y_hbm, src_hbm, dst_hbm, z640_hbm, agg_out,
                   si128, di128, gidx, sidx, gbuf, aggsh):
    c = lax.axis_index("c")
    s = lax.axis_index("s")
    wid = _wid()

    for q in range(4):
        pltpu.sync_copy(z640_hbm, aggsh.at[pl.ds(s * 2560 + q * 640, 640)])
    plsc.subcore_barrier()

    def blk_body(j, carry):
        sub = wid + NW * j

        @pl.when(sub < NSUB)
        def _():
            base = sub * 128
            pltpu.sync_copy(src_hbm.at[pl.ds(base, 128)], si128)
            pltpu.sync_copy(dst_hbm.at[pl.ds(base, 128)], di128)
            for cc in range(4):
                for v in range(8):
                    sv = si128[pl.ds(v * 16, 16)]
                    dv = di128[pl.ds(v * 16, 16)]
                    gidx[pl.ds(v * 16, 16)] = sv * 4 + cc
                    sidx[pl.ds(v * 16, 16)] = dv * 4 + cc
                pltpu.sync_copy(y_hbm.at[gidx], gbuf)
                pltpu.sync_copy(gbuf, aggsh.at[sidx], add=True)

        return carry

    lax.fori_loop(0, (NSUB + NW - 1) // NW, blk_body, 0)
    plsc.subcore_barrier()
    pltpu.sync_copy(aggsh.at[pl.ds(s * 2560, 2560)],
                    agg_out.at[pl.ds(c * NPAD0 * 4 + s * 2560, 2560)])


def _gcn0_agg(y, src, dst):
    z640 = jnp.zeros((640,), _f32)
    kern = pl.kernel(
        _gcn0_agg_body,
        out_type=jax.ShapeDtypeStruct((NC * NPAD0 * 4,), _f32),
        mesh=_sc_mesh(),
        compiler_params=pltpu.CompilerParams(needs_layout_passes=False),
        scratch_types=[
            pltpu.VMEM((128,), _i32), pltpu.VMEM((128,), _i32),
            pltpu.VMEM((128,), _i32), pltpu.VMEM((128,), _i32),
            pltpu.VMEM((128,), _f32),
            pltpu.VMEM_SHARED((NPAD0 * 4,), _f32),
        ],
        name="gcn0_agg",
    )
    return kern(y.reshape(NPAD0 * 4), src, dst, z640)


def _gat_edge_body(kpad, kd, buckets,
                   comps, compd, cnt_hbm, als_hbm, ald_hbm, m_hbm, h_hbm,
                   znum_hbm, zden_hbm,
                   num_out, den_out,
                   alsv, aldv, mv, sbufw, dbufw, wbuf, rowbuf,
                   cntv, hsh, numsh, densh):
    c = lax.axis_index("c")
    s = lax.axis_index("s")
    wid = _wid()
    rows_h = kpad // 16
    rows_t = kd // 16
    dump = kpad

    nchd = kd // 128
    pltpu.sync_copy(als_hbm, alsv)
    pltpu.sync_copy(ald_hbm, aldv)
    pltpu.sync_copy(m_hbm, mv)
    pltpu.sync_copy(cnt_hbm.at[pl.ds(wid * 128, 128)], cntv)
    pltpu.sync_copy(h_hbm.at[pl.ds(s * rows_h, rows_h)],
                    hsh.at[pl.ds(s * rows_h, rows_h)])
    pltpu.sync_copy(znum_hbm, numsh.at[pl.ds(s * rows_t, rows_t)])
    for i in range((nchd + NS - 1) // NS):
        chunk = s + NS * i

        @pl.when(chunk < nchd)
        def _(chunk=chunk):
            pltpu.sync_copy(zden_hbm, densh.at[pl.ds(chunk * 128, 128)])

    plsc.subcore_barrier()

    cntvec = cntv[pl.ds(0, 16)]
    for b in buckets:
        cb = cntvec[b]
        segbase = (b * NW + wid) * PTP

        def chunk_body(j, carry, cb=cb, segbase=segbase):
            @pl.when(j * 128 < cb)
            def _():
                pltpu.sync_copy(comps.at[pl.ds(segbase + j * 128, 128)], sbufw)
                pltpu.sync_copy(compd.at[pl.ds(segbase + j * 128, 128)], dbufw)
                for v in range(8):
                    sv = sbufw[pl.ds(v * 16, 16)]
                    dv = dbufw[pl.ds(v * 16, 16)]
                    lanepos = j * 128 + v * 16 + lax.iota(_i32, 16)
                    mvld = lanepos < cb
                    s_c = jnp.where(mvld, sv, 0)
                    d_t = jnp.where(mvld, dv, 0)
                    a1 = plsc.load_gather(alsv, [s_c])
                    a2 = plsc.load_gather(aldv, [d_t])
                    mm = plsc.load_gather(mv, [d_t])
                    e = a1 + a2
                    e = jnp.where(e > 0, e, 0.2 * e)
                    w = jnp.where(mvld, jnp.exp(e - mm), 0.0)
                    wbuf[pl.ds(v * 16, 16)] = w
                    sbufw[pl.ds(v * 16, 16)] = s_c
                    dbufw[pl.ds(v * 16, 16)] = jnp.where(mvld, dv, dump)
                pltpu.sync_copy(wbuf.at[pl.ds(0, 128)], densh.at[dbufw],
                                add=True)
                pltpu.sync_copy(hsh.at[sbufw], rowbuf)

                def scale_row(r, carry2):
                    wr = wbuf[pl.ds(r, 16)][0]
                    for cc in range(8):
                        rowbuf[r, pl.ds(cc * 16, 16)] = (
                            rowbuf[r, pl.ds(cc * 16, 16)] * wr)
                    return carry2

                lax.fori_loop(0, 128, scale_row, 0)
                pltpu.sync_copy(rowbuf, numsh.at[dbufw], add=True)

            return carry

        lax.fori_loop(0, NCH, chunk_body, 0)

    plsc.subcore_barrier()
    pltpu.sync_copy(numsh.at[pl.ds(s * rows_t, rows_t)],
                    num_out.at[pl.ds(c * kd + s * rows_t, rows_t)])
    for i in range((nchd + NS - 1) // NS):
        chunk = s + NS * i

        @pl.when(chunk < nchd)
        def _(chunk=chunk):
            pltpu.sync_copy(densh.at[pl.ds(chunk * 128, 128)],
                            den_out.at[pl.ds(c * kd + chunk * 128, 128)])


def _gat_edge(kpad, kd, buckets, comps, compd, cnt, als, ald, m, h):
    rows_t = kd // 16
    znum = jnp.zeros((rows_t, 128), _f32)
    zden = jnp.zeros((128,), _f32)
    kern = pl.kernel(
        functools.partial(_gat_edge_body, kpad, kd, buckets),
        out_type=(
            jax.ShapeDtypeStruct((NC * kd, 128), _f32),
            jax.ShapeDtypeStruct((NC * kd,), _f32),
        ),
        mesh=_sc_mesh(),
        compiler_params=pltpu.CompilerParams(needs_layout_passes=False),
        scratch_types=[
            pltpu.VMEM((kpad,), _f32), pltpu.VMEM((kpad,), _f32),
            pltpu.VMEM((kpad,), _f32),
            pltpu.VMEM((128,), _i32), pltpu.VMEM((128,), _i32),
            pltpu.VMEM((144,), _f32), pltpu.VMEM((128, 128), _f32),
            pltpu.VMEM((128,), _i32),
            pltpu.VMEM_SHARED((kpad, 128), _f32),
            pltpu.VMEM_SHARED((kd, 128), _f32),
            pltpu.VMEM_SHARED((kd,), _f32),
        ],
        name=f"gat_edge_{kpad}",
    )
    return kern(comps, compd, cnt, als, ald, m, h, znum, zden)


def _dec_agg_body(y_hbm, comps, compd, cnt_hbm, zrow_hbm, agg_out,
                  sbufw, dbufw, rowbuf, cntv, ysh, aggsh):
    c = lax.axis_index("c")
    s = lax.axis_index("s")
    wid = _wid()
    rows_y = KP3 // 16
    rows_t = KD3 // 16

    pltpu.sync_copy(cnt_hbm.at[pl.ds(wid * 128, 128)], cntv)
    pltpu.sync_copy(y_hbm.at[pl.ds(s * rows_y, rows_y)],
                    ysh.at[pl.ds(s * rows_y, rows_y)])
    pltpu.sync_copy(zrow_hbm, aggsh.at[pl.ds(s * rows_t, rows_t)])
    plsc.subcore_barrier()

    cb = cntv[pl.ds(0, 16)][0]
    segbase = wid * PTP

    def chunk_body(j, carry):
        @pl.when(j * 128 < cb)
        def _():
            pltpu.sync_copy(comps.at[pl.ds(segbase + j * 128, 128)], sbufw)
            pltpu.sync_copy(compd.at[pl.ds(segbase + j * 128, 128)], dbufw)
            for v in range(8):
                sv = sbufw[pl.ds(v * 16, 16)]
                dv = dbufw[pl.ds(v * 16, 16)]
                lanepos = j * 128 + v * 16 + lax.iota(_i32, 16)
                mvld = lanepos < cb
                sbufw[pl.ds(v * 16, 16)] = jnp.where(mvld, sv, 0)
                dbufw[pl.ds(v * 16, 16)] = jnp.where(mvld, dv, KP3)
            pltpu.sync_copy(ysh.at[sbufw], rowbuf)
            pltpu.sync_copy(rowbuf, aggsh.at[dbufw], add=True)

        return carry

    lax.fori_loop(0, NCH, chunk_body, 0)
    plsc.subcore_barrier()
    pltpu.sync_copy(aggsh.at[pl.ds(s * rows_t, rows_t)],
                    agg_out.at[pl.ds(c * KD3 + s * rows_t, rows_t)])


def _dec_agg(y, comps, compd, cnt):
    zrow = jnp.zeros((KD3 // 16, 128), _f32)
    kern = pl.kernel(
        _dec_agg_body,
        out_type=jax.ShapeDtypeStruct((NC * KD3, 128), _f32),
        mesh=_sc_mesh(),
        compiler_params=pltpu.CompilerParams(needs_layout_passes=False),
        scratch_types=[
            pltpu.VMEM((128,), _i32), pltpu.VMEM((128,), _i32),
            pltpu.VMEM((128, 128), _f32),
            pltpu.VMEM((128,), _i32),
            pltpu.VMEM_SHARED((KP3, 128), _f32),
            pltpu.VMEM_SHARED((KD3, 128), _f32),
        ],
        name="dec_agg",
    )
    return kern(y, comps, compd, cnt, zrow)


def _prep0_body(degT_ref, x4_ref, y_ref, dis_ref):
    degT = degT_ref[...]
    d = degT[:, 0:1] + degT[:, 1:2] + 1.0
    dis = lax.rsqrt(d)
    dis_ref[...] = dis
    y_ref[...] = dis * x4_ref[...]


def _prep0(degT, x4):
    return pl.pallas_call(
        _prep0_body,
        out_shape=(jax.ShapeDtypeStruct((NPAD0, 4), _f32),
                   jax.ShapeDtypeStruct((NPAD0, 1), _f32)),
    )(degT, x4)


def _gcn0_fin_body(agg_ref, x4_ref, dis_ref, W4_ref, b_ref, p_ref,
                   h_ref, score_ref):
    dis = dis_ref[...]
    agg = agg_ref[...]
    aggs = agg[0] + agg[1]
    pre = dis * aggs + (dis * dis) * x4_ref[...]
    h = jnp.dot(pre, W4_ref[...], preferred_element_type=_f32) + b_ref[...]
    h_ref[...] = h
    p = p_ref[...]
    pn = p * lax.rsqrt(jnp.sum(p * p))
    proj = jnp.dot(h, pn, preferred_element_type=_f32)
    rows = lax.broadcasted_iota(_i32, (NPAD0, 1), 0)
    score_ref[...] = jnp.where(rows < N0, jnp.tanh(proj), -1e30)


def _gcn0_fin(agg, x4, dis, W4, b, p):
    return pl.pallas_call(
        _gcn0_fin_body,
        out_shape=(jax.ShapeDtypeStruct((NPAD0, H), _f32),
                   jax.ShapeDtypeStruct((NPAD0, 1), _f32)),
    )(agg, x4, dis, W4, b, p)


def _rank_body(npad, scol_ref, srow_ref, out_ref):
    ib = pl.program_id(0)
    si = scol_ref[...]
    i_ids = ib * 512 + lax.broadcasted_iota(_i32, (512, 1), 0)

    def body(j, acc):
        sj = srow_ref[:, pl.ds(j * 512, 512)]
        j_ids = j * 512 + lax.broadcasted_iota(_i32, (1, 512), 1)
        gt = (sj > si).astype(_i32)
        eq = ((sj == si) & (j_ids < i_ids)).astype(_i32)
        return acc + jnp.sum(gt + eq, axis=1, keepdims=True)

    out_ref[...] = lax.fori_loop(0, npad // 512, body,
                                 jnp.zeros((512, 1), _i32))


def _rank(score_col, npad):
    srow = score_col.reshape(1, npad)
    return pl.pallas_call(
        functools.partial(_rank_body, npad),
        grid=(npad // 512,),
        in_specs=[pl.BlockSpec((512, 1), lambda i: (i, 0)),
                  pl.BlockSpec((1, npad), lambda i: (0, 0))],
        out_specs=pl.BlockSpec((512, 1), lambda i: (i, 0)),
        out_shape=jax.ShapeDtypeStruct((npad, 1), _i32),
    )(score_col, srow)


def _select_body(npad, k, rank_ref, srow_ref, h_ref, hsel_ref, vals_ref):
    rb = pl.program_id(0)
    r_ids = rb * 256 + lax.broadcasted_iota(_i32, (256, 1), 0)

    def body(ic, carry):
        acc, vacc = carry
        rk = rank_ref[:, pl.ds(ic * 512, 512)]
        sc = srow_ref[:, pl.ds(ic * 512, 512)]
        hit = (rk == r_ids) & (r_ids < k)
        P = jnp.where(hit, sc, 0.0)
        acc = acc + jnp.dot(P, h_ref[pl.ds(ic * 512, 512), :],
                            preferred_element_type=_f32)
        vacc = vacc + jnp.sum(P, axis=1, keepdims=True)
        return acc, vacc

    acc, vacc = lax.fori_loop(
        0, npad // 512, body,
        (jnp.zeros((256, H), _f32), jnp.zeros((256, 1), _f32)))
    hsel_ref[...] = acc
    vals_ref[...] = vacc


def _select(rank_col, score_col, h, npad, k, kpad):
    rrow = rank_col.reshape(1, npad)
    srow = score_col.reshape(1, npad)
    return pl.pallas_call(
        functools.partial(_select_body, npad, k),
        grid=(kpad // 256,),
        in_specs=[pl.BlockSpec((1, npad), lambda i: (0, 0)),
                  pl.BlockSpec((1, npad), lambda i: (0, 0)),
                  pl.BlockSpec((npad, H), lambda i: (0, 0))],
        out_specs=(pl.BlockSpec((256, H), lambda i: (i, 0)),
                   pl.BlockSpec((256, 1), lambda i: (i, 0))),
        out_shape=(jax.ShapeDtypeStruct((kpad, H), _f32),
                   jax.ShapeDtypeStruct((kpad, 1), _f32)),
    )(rrow, srow, h)


def _gat_dense_body(n, kpad, hsel_ref, W_ref, as_ref, ad_ref,
                    h_ref, als_ref, ald_ref, m_ref, ws_ref):
    rows = lax.broadcasted_iota(_i32, (kpad, 1), 0)
    h1 = hsel_ref[...]
    h = jnp.dot(h1, W_ref[...], preferred_element_type=_f32)
    h_ref[...] = h
    als = jnp.dot(h, as_ref[...], preferred_element_type=_f32)
    ald = jnp.dot(h, ad_ref[...], preferred_element_type=_f32)
    als_ref[...] = als
    ald_ref[...] = ald
    amax = jnp.max(jnp.where(rows < n, als, -3e38))
    pre_m = amax + ald
    m = jnp.where(pre_m > 0, pre_m, 0.2 * pre_m)
    m_ref[...] = m
    pre_s = als + ald
    es = jnp.where(pre_s > 0, pre_s, 0.2 * pre_s)
    ws_ref[...] = jnp.maximum(jnp.exp(es - m), 1e-30)


def _gat_dense(hsel, W, a_s, a_d, n, kpad):
    return pl.pallas_call(
        functools.partial(_gat_dense_body, n, kpad),
        out_shape=(jax.ShapeDtypeStruct((kpad, H), _f32),
                   jax.ShapeDtypeStruct((kpad, 1), _f32),
                   jax.ShapeDtypeStruct((kpad, 1), _f32),
                   jax.ShapeDtypeStruct((kpad, 1), _f32),
                   jax.ShapeDtypeStruct((kpad, 1), _f32)),
    )(hsel, W, a_s, a_d)


def _gat_fin_body(n, kpad, num_ref, den_ref, ws_ref, h_ref, b_ref, p_ref,
                  hout_ref, score_ref):
    ws = ws_ref[...]
    num_a = num_ref[...]
    den_a = den_ref[...]
    num = (num_a[0, :kpad, :] + num_a[1, :kpad, :] + ws * h_ref[...])
    den = (den_a[0, :kpad, :] + den_a[1, :kpad, :] + ws)
    out = num / den + b_ref[...]
    hout_ref[...] = out
    p = p_ref[...]
    pn = p * lax.rsqrt(jnp.sum(p * p))
    proj = jnp.dot(out, pn, preferred_element_type=_f32)
    rows = lax.broadcasted_iota(_i32, (kpad, 1), 0)
    score_ref[...] = jnp.where(rows < n, jnp.tanh(proj), -1e30)


def _gat_fin(num, den3, ws, h, b, p, n, kpad):
    return pl.pallas_call(
        functools.partial(_gat_fin_body, n, kpad),
        out_shape=(jax.ShapeDtypeStruct((kpad, H), _f32),
                   jax.ShapeDtypeStruct((kpad, 1), _f32)),
    )(num, den3, ws, h, b, p)


def _vae_body(degT_ref, hsel_ref, Wmu_ref, bmu_ref, Wlv_ref, blv_ref,
              Wld_ref, bld_ref, eps_ref,
              mu_ref, lv_ref, z_ref, y_ref, dis_ref):
    h5 = hsel_ref[...]
    mu = jnp.dot(h5, Wmu_ref[...], preferred_element_type=_f32) + bmu_ref[...]
    lv = jnp.dot(h5, Wlv_ref[...], preferred_element_type=_f32) + blv_ref[...]
    mu_ref[...] = mu
    lv_ref[...] = lv
    zlat = mu + eps_ref[...] * jnp.exp(0.5 * lv)
    z = jnp.dot(zlat, Wld_ref[...], preferred_element_type=_f32) + bld_ref[...]
    z_ref[...] = z
    degT = degT_ref[...]
    d = degT[:, 0:1] + degT[:, 1:2] + 2.0
    dis = lax.rsqrt(d)
    dis_ref[...] = dis
    y_ref[...] = dis * z


def _vae(degT, hsel, Wmu, bmu, Wlv, blv, Wld, bld, eps):
    return pl.pallas_call(
        _vae_body,
        out_shape=(jax.ShapeDtypeStruct((KP3, LAT), _f32),
                   jax.ShapeDtypeStruct((KP3, LAT), _f32),
                   jax.ShapeDtypeStruct((KP3, H), _f32),
                   jax.ShapeDtypeStruct((KP3, H), _f32),
                   jax.ShapeDtypeStruct((KP3, 1), _f32)),
    )(degT, hsel, Wmu, bmu, Wlv, blv, Wld, bld, eps)


def _dec_fin_body(agg_ref, z_ref, dis_ref, W_ref, b_ref, zn_ref, yn_ref):
    dis = dis_ref[...]
    agg = agg_ref[...]
    aggs = agg[0, :KP3, :] + agg[1, :KP3, :]
    pre = dis * aggs + 2.0 * (dis * dis) * z_ref[...]
    zn = jnp.dot(pre, W_ref[...], preferred_element_type=_f32) + b_ref[...]
    zn_ref[...] = zn
    yn_ref[...] = dis * zn


def _dec_fin(agg, z, dis, W, b):
    return pl.pallas_call(
        _dec_fin_body,
        out_shape=(jax.ShapeDtypeStruct((KP3, H), _f32),
                   jax.ShapeDtypeStruct((KP3, H), _f32)),
    )(agg, z, dis, W, b)


def kernel(x, edge_index, undirected_edge_index, batch, params):
    p = params
    src = edge_index[0]
    dst = edge_index[1]
    src2d = src.reshape(NSUB, 128)
    dst2d = dst.reshape(NSUB, 128)

    x4 = jnp.pad(x, ((0, NPAD0 - N0), (0, 1)))
    W4 = jnp.pad(p['W_e0'], ((0, 1), (0, 0)))

    deg, degdec, comps, compd, cnt = _edge_prep(src, dst)

    y0, dis0 = _prep0(deg.reshape(NC, NPAD0).T, x4)
    agg0 = _gcn0_agg(y0, src, dst).reshape(NC, NPAD0, 4)
    h0, score0 = _gcn0_fin(agg0, x4, dis0, W4, p['b_e0'].reshape(1, H),
                           p['p0'].reshape(H, 1))

    rk0 = _rank(score0, NPAD0)
    hsel1, _ = _select(rk0, score0, h0, NPAD0, K1, KP1)
    hg1, als1, ald1, m1, ws1 = _gat_dense(hsel1, p['W_e1'],
                                          p['a_src1'].reshape(H, 1),
                                          p['a_dst1'].reshape(H, 1), K1, KP1)
    num1, den1 = _gat_edge(KP1, KD1, (0, 1, 2), comps, compd, cnt,
                           als1.reshape(KP1), ald1.reshape(KP1),
                           m1.reshape(KP1), hg1)
    h1, score1 = _gat_fin(num1.reshape(NC, KD1, H),
                          den1.reshape(NC, KD1, 1), ws1, hg1,
                          p['b_e1'].reshape(1, H), p['p1'].reshape(H, 1),
                          K1, KP1)

    rk1 = _rank(score1, KP1)
    hsel2, _ = _select(rk1, score1, h1, KP1, K2, KP2)
    hg2, als2, ald2, m2, ws2 = _gat_dense(hsel2, p['W_e2'],
                                          p['a_src2'].reshape(H, 1),
                                          p['a_dst2'].reshape(H, 1), K2, KP2)
    num2, den2 = _gat_edge(KP2, KD2, (0, 1), comps, compd, cnt,
                           als2.reshape(KP2), ald2.reshape(KP2),
                           m2.reshape(KP2), hg2)
    h2, score2 = _gat_fin(num2.reshape(NC, KD2, H),
                          den2.reshape(NC, KD2, 1), ws2, hg2,
                          p['b_e2'].reshape(1, H), p['p2'].reshape(H, 1),
                          K2, KP2)

    rk2 = _rank(score2, KP2)
    hsel3, _ = _select(rk2, score2, h2, KP2, K3, KP3)
    eps = jax.random.normal(jax.random.key(42), (K3, LAT), _f32)
    eps = jnp.pad(eps, ((0, KP3 - K3), (0, 0)))
    degdecT = degdec.reshape(NC, KD3).T[:KP3, :]
    mu, lv, z, y, disd = _vae(degdecT, hsel3,
                              p['W_mu'], p['b_mu'].reshape(1, LAT),
                              p['W_lv'], p['b_lv'].reshape(1, LAT),
                              p['W_ld'], p['b_ld'].reshape(1, H), eps)

    for Wd, bd in [(p['W_d2'], p['b_d2']), (p['W_d1'], p['b_d1']),
                   (p['W_d0'], p['b_d0'])]:
        aggd = _dec_agg(y, comps, compd, cnt).reshape(NC, KD3, H)
        z, y = _dec_fin(aggd, z, disd, Wd, bd.reshape(1, H))

    return z[:K3], mu[:K3], lv[:K3]


# speedup vs baseline: 168.7934x; 1.2298x over previous
import functools

import jax
import jax.numpy as jnp
from jax import lax
from jax.experimental import pallas as pl
from jax.experimental.pallas import tpu as pltpu
from jax.experimental.pallas import tpu_sc as plsc

NC, NS = 2, 16
NW = NC * NS
N0, E, H, LAT = 10000, 640000, 128, 16
NPAD0 = 10240
PT = E // NW
ECH = 2560
NECH = E // ECH
PTP = 20608
NCH = PTP // 128
NSUB = E // 128
NBLK8 = E // (128 * 8)

K1, K2, K3 = 5000, 2500, 1250
KP1, KP2, KP3 = 5120, 2560, 1280
KD1, KD2, KD3 = 5248, 2688, 1408
CDUMP = 3 * NW * PTP

_f32 = jnp.float32
_i32 = jnp.int32


def _sc_mesh():
    return plsc.VectorSubcoreMesh(core_axis_name="c", subcore_axis_name="s")


def _wid():
    return lax.axis_index("s") * NC + lax.axis_index("c")


_SPD = NS * 3 * 2 * PTP


def _edge_prep_body(src_hbm, dst_hbm, one_hbm, z640_hbm,
                    deg_out, degdec_out, comps_out, compd_out, cnt_out,
                    sbuf, dbuf, pidx, pdidx, degidx, ddidx, ddval, one128,
                    cntv, degsh, ddsh, spcomp):
    c = lax.axis_index("c")
    s = lax.axis_index("s")
    wid = _wid()

    pltpu.sync_copy(one_hbm, one128)
    pltpu.sync_copy(z640_hbm, degsh.at[pl.ds(s * 640, 640)])

    @pl.when(s < KD3 // 128)
    def _():
        pltpu.sync_copy(z640_hbm.at[pl.ds(0, 128)], ddsh.at[pl.ds(s * 128, 128)])

    plsc.subcore_barrier()

    lanes16 = lax.iota(_i32, 16)
    baseA = (s * 3 + 0) * 2 * PTP
    baseB = (s * 3 + 1) * 2 * PTP
    baseC = (s * 3 + 2) * 2 * PTP
    dumpbase = _SPD + s * 128

    def chunk_body(i, offs):
        ch = wid + NW * i

        def do_chunk(offs_in):
            pltpu.sync_copy(src_hbm.at[pl.ds(ch * ECH, ECH)], sbuf)
            pltpu.sync_copy(dst_hbm.at[pl.ds(ch * ECH, ECH)], dbuf)

            def grp_body(g, offs2):
                offA, offB, offC = offs2
                for v in range(8):
                    sv = sbuf[pl.ds(g * 128 + v * 16, 16)]
                    dv = dbuf[pl.ds(g * 128 + v * 16, 16)]
                    mx = jnp.maximum(sv, dv)
                    mA = mx < K3
                    mB = (mx >= K3) & (mx < K2)
                    mC = (mx >= K2) & (mx < K1)
                    valid = mx < K1
                    csA = plsc.cumsum(mA.astype(_i32))
                    csB = plsc.cumsum(mB.astype(_i32))
                    csC = plsc.cumsum(mC.astype(_i32))
                    dumpv = dumpbase + v * 16 + lanes16
                    posS = jnp.where(
                        mA, baseA + offA + csA - 1,
                        jnp.where(mB, baseB + offB + csB - 1,
                                  jnp.where(mC, baseC + offC + csC - 1,
                                            dumpv)))
                    posD = posS + jnp.where(valid, PTP, 0)
                    pidx[pl.ds(v * 16, 16)] = posS
                    pdidx[pl.ds(v * 16, 16)] = posD
                    degidx[pl.ds(v * 16, 16)] = dv
                    ddidx[pl.ds(v * 16, 16)] = jnp.where(mA, dv, KP3)
                    ddval[pl.ds(v * 16, 16)] = jnp.where(mA, 1.0, 0.0).astype(_f32)
                    offA = offA + csA[15]
                    offB = offB + csB[15]
                    offC = offC + csC[15]
                vals_s = sbuf.at[pl.ds(g * 128, 128)]
                vals_d = dbuf.at[pl.ds(g * 128, 128)]
                pltpu.sync_copy(vals_s, spcomp.at[pidx])
                pltpu.sync_copy(vals_d, spcomp.at[pdidx])
                pltpu.sync_copy(one128, degsh.at[degidx], add=True)
                pltpu.sync_copy(ddval, ddsh.at[ddidx], add=True)
                return offA, offB, offC

            return lax.fori_loop(0, ECH // 128, grp_body, offs_in)

        return lax.cond(ch < NECH, do_chunk, lambda o: o, offs)

    z = jnp.int32(0)
    offA, offB, offC = lax.fori_loop(0, (NECH + NW - 1) // NW, chunk_body,
                                     (z, z, z))

    for b, base in enumerate((baseA, baseB, baseC)):
        seg = (b * NW + wid) * PTP
        pltpu.sync_copy(spcomp.at[pl.ds(base, PTP)],
                        comps_out.at[pl.ds(seg, PTP)])
        pltpu.sync_copy(spcomp.at[pl.ds(base + PTP, PTP)],
                        compd_out.at[pl.ds(seg, PTP)])

    lanes = lax.iota(_i32, 16)
    cvec = jnp.zeros((16,), _i32)
    for b, off_b in enumerate((offA, offB, offC)):
        cvec = jnp.where(lanes == b, jnp.zeros((16,), _i32) + off_b, cvec)
    cntv[pl.ds(0, 16)] = cvec
    for t in range(1, 8):
        cntv[pl.ds(t * 16, 16)] = jnp.zeros((16,), _i32)
    pltpu.sync_copy(cntv, cnt_out.at[pl.ds(wid * 128, 128)])

    plsc.subcore_barrier()

    @pl.when(s == 0)
    def _():
        pltpu.sync_copy(degsh, deg_out.at[pl.ds(c * NPAD0, NPAD0)])
        pltpu.sync_copy(ddsh, degdec_out.at[pl.ds(c * KD3, KD3)])


def _edge_prep(src, dst):
    ones = jnp.ones((128,), _f32)
    z640 = jnp.zeros((640,), _f32)
    kern = pl.kernel(
        _edge_prep_body,
        out_type=(
            jax.ShapeDtypeStruct((NC * NPAD0,), _f32),
            jax.ShapeDtypeStruct((NC * KD3,), _f32),
            jax.ShapeDtypeStruct((CDUMP,), _i32),
            jax.ShapeDtypeStruct((CDUMP,), _i32),
            jax.ShapeDtypeStruct((NW * 128,), _i32),
        ),
        mesh=_sc_mesh(),
        compiler_params=pltpu.CompilerParams(needs_layout_passes=False),
        scratch_types=[
            pltpu.VMEM((ECH,), _i32), pltpu.VMEM((ECH,), _i32),
            pltpu.VMEM((128,), _i32), pltpu.VMEM((128,), _i32),
            pltpu.VMEM((128,), _i32), pltpu.VMEM((128,), _i32),
            pltpu.VMEM((128,), _f32), pltpu.VMEM((128,), _f32),
            pltpu.VMEM((128,), _i32),
            pltpu.VMEM_SHARED((NPAD0,), _f32),
            pltpu.VMEM_SHARED((KD3,), _f32),
            pltpu.VMEM_SHARED((_SPD + NS * 128,), _i32),
        ],
        name="edge_prep",
    )
    return kern(src, dst, ones, z640)


def _gcn0_agg_body(y_hbm, src_hbm, dst_hbm, z640_hbm, agg_out,
                   si128, di128, gidx0, gidx1, gidx2, gidx3,
                   sidx0, sidx1, sidx2, sidx3,
                   gbuf0, gbuf1, gbuf2, gbuf3, semg, sems, aggsh):
    c = lax.axis_index("c")
    s = lax.axis_index("s")
    wid = _wid()
    gidx = (gidx0, gidx1, gidx2, gidx3)
    sidx = (sidx0, sidx1, sidx2, sidx3)
    gbuf = (gbuf0, gbuf1, gbuf2, gbuf3)

    for q in range(4):
        pltpu.sync_copy(z640_hbm, aggsh.at[pl.ds(s * 2560 + q * 640, 640)])
    plsc.subcore_barrier()

    def blk_body(j, carry):
        sub = wid + NW * j

        @pl.when(sub < NSUB)
        def _():
            base = sub * 128
            d1 = pltpu.async_copy(src_hbm.at[pl.ds(base, 128)], si128, semg)
            d2 = pltpu.async_copy(dst_hbm.at[pl.ds(base, 128)], di128, semg)
            d1.wait()
            d2.wait()
            for cc in range(4):
                for v in range(8):
                    sv = si128[pl.ds(v * 16, 16)]
                    dv = di128[pl.ds(v * 16, 16)]
                    gidx[cc][pl.ds(v * 16, 16)] = sv * 4 + cc
                    sidx[cc][pl.ds(v * 16, 16)] = dv * 4 + cc
            descs = [pltpu.async_copy(y_hbm.at[gidx[cc]], gbuf[cc], semg)
                     for cc in range(4)]
            for d in descs:
                d.wait()
            descs = [pltpu.async_copy(gbuf[cc], aggsh.at[sidx[cc]], sems,
                                      add=True)
                     for cc in range(4)]
            for d in descs:
                d.wait()

        return carry

    lax.fori_loop(0, (NSUB + NW - 1) // NW, blk_body, 0)
    plsc.subcore_barrier()
    pltpu.sync_copy(aggsh.at[pl.ds(s * 2560, 2560)],
                    agg_out.at[pl.ds(c * NPAD0 * 4 + s * 2560, 2560)])


def _gcn0_agg(y, src, dst):
    z640 = jnp.zeros((640,), _f32)
    kern = pl.kernel(
        _gcn0_agg_body,
        out_type=jax.ShapeDtypeStruct((NC * NPAD0 * 4,), _f32),
        mesh=_sc_mesh(),
        compiler_params=pltpu.CompilerParams(needs_layout_passes=False),
        scratch_types=[
            pltpu.VMEM((128,), _i32), pltpu.VMEM((128,), _i32),
            pltpu.VMEM((128,), _i32), pltpu.VMEM((128,), _i32),
            pltpu.VMEM((128,), _i32), pltpu.VMEM((128,), _i32),
            pltpu.VMEM((128,), _i32), pltpu.VMEM((128,), _i32),
            pltpu.VMEM((128,), _i32), pltpu.VMEM((128,), _i32),
            pltpu.VMEM((128,), _f32), pltpu.VMEM((128,), _f32),
            pltpu.VMEM((128,), _f32), pltpu.VMEM((128,), _f32),
            pltpu.SemaphoreType.DMA, pltpu.SemaphoreType.DMA,
            pltpu.VMEM_SHARED((NPAD0 * 4,), _f32),
        ],
        name="gcn0_agg",
    )
    return kern(y.reshape(NPAD0 * 4), src, dst, z640)


def _gat_edge_body(kpad, kd, buckets,
                   comps, compd, cnt_hbm, als_hbm, ald_hbm, m_hbm, h_hbm,
                   znum_hbm, zden_hbm,
                   num_out, den_out,
                   alsv, aldv, mv, sbufw, dbufw, wbuf, rowbuf,
                   cntv, hsh, numsh, densh):
    c = lax.axis_index("c")
    s = lax.axis_index("s")
    wid = _wid()
    rows_h = kpad // 16
    rows_t = kd // 16
    dump = kpad

    nchd = kd // 128
    pltpu.sync_copy(als_hbm, alsv)
    pltpu.sync_copy(ald_hbm, aldv)
    pltpu.sync_copy(m_hbm, mv)
    pltpu.sync_copy(cnt_hbm.at[pl.ds(wid * 128, 128)], cntv)
    pltpu.sync_copy(h_hbm.at[pl.ds(s * rows_h, rows_h)],
                    hsh.at[pl.ds(s * rows_h, rows_h)])
    pltpu.sync_copy(znum_hbm, numsh.at[pl.ds(s * rows_t, rows_t)])
    for i in range((nchd + NS - 1) // NS):
        chunk = s + NS * i

        @pl.when(chunk < nchd)
        def _(chunk=chunk):
            pltpu.sync_copy(zden_hbm, densh.at[pl.ds(chunk * 128, 128)])

    plsc.subcore_barrier()

    cntvec = cntv[pl.ds(0, 16)]
    for b in buckets:
        cb = cntvec[b]
        segbase = (b * NW + wid) * PTP

        def chunk_body(j, carry, cb=cb, segbase=segbase):
            @pl.when(j * 128 < cb)
            def _():
                pltpu.sync_copy(comps.at[pl.ds(segbase + j * 128, 128)], sbufw)
                pltpu.sync_copy(compd.at[pl.ds(segbase + j * 128, 128)], dbufw)
                for v in range(8):
                    sv = sbufw[pl.ds(v * 16, 16)]
                    dv = dbufw[pl.ds(v * 16, 16)]
                    lanepos = j * 128 + v * 16 + lax.iota(_i32, 16)
                    mvld = lanepos < cb
                    s_c = jnp.where(mvld, sv, 0)
                    d_t = jnp.where(mvld, dv, 0)
                    a1 = plsc.load_gather(alsv, [s_c])
                    a2 = plsc.load_gather(aldv, [d_t])
                    mm = plsc.load_gather(mv, [d_t])
                    e = a1 + a2
                    e = jnp.where(e > 0, e, 0.2 * e)
                    w = jnp.where(mvld, jnp.exp(e - mm), 0.0)
                    wbuf[pl.ds(v * 16, 16)] = w
                    sbufw[pl.ds(v * 16, 16)] = s_c
                    dbufw[pl.ds(v * 16, 16)] = jnp.where(mvld, dv, dump)
                pltpu.sync_copy(wbuf.at[pl.ds(0, 128)], densh.at[dbufw],
                                add=True)
                pltpu.sync_copy(hsh.at[sbufw], rowbuf)

                def scale_row(r, carry2):
                    wr = wbuf[pl.ds(r, 16)][0]
                    for cc in range(8):
                        rowbuf[r, pl.ds(cc * 16, 16)] = (
                            rowbuf[r, pl.ds(cc * 16, 16)] * wr)
                    return carry2

                lax.fori_loop(0, 128, scale_row, 0)
                pltpu.sync_copy(rowbuf, numsh.at[dbufw], add=True)

            return carry

        lax.fori_loop(0, NCH, chunk_body, 0)

    plsc.subcore_barrier()
    pltpu.sync_copy(numsh.at[pl.ds(s * rows_t, rows_t)],
                    num_out.at[pl.ds(c * kd + s * rows_t, rows_t)])
    for i in range((nchd + NS - 1) // NS):
        chunk = s + NS * i

        @pl.when(chunk < nchd)
        def _(chunk=chunk):
            pltpu.sync_copy(densh.at[pl.ds(chunk * 128, 128)],
                            den_out.at[pl.ds(c * kd + chunk * 128, 128)])


def _gat_edge(kpad, kd, buckets, comps, compd, cnt, als, ald, m, h):
    rows_t = kd // 16
    znum = jnp.zeros((rows_t, 128), _f32)
    zden = jnp.zeros((128,), _f32)
    kern = pl.kernel(
        functools.partial(_gat_edge_body, kpad, kd, buckets),
        out_type=(
            jax.ShapeDtypeStruct((NC * kd, 128), _f32),
            jax.ShapeDtypeStruct((NC * kd,), _f32),
        ),
        mesh=_sc_mesh(),
        compiler_params=pltpu.CompilerParams(needs_layout_passes=False),
        scratch_types=[
            pltpu.VMEM((kpad,), _f32), pltpu.VMEM((kpad,), _f32),
            pltpu.VMEM((kpad,), _f32),
            pltpu.VMEM((128,), _i32), pltpu.VMEM((128,), _i32),
            pltpu.VMEM((144,), _f32), pltpu.VMEM((128, 128), _f32),
            pltpu.VMEM((128,), _i32),
            pltpu.VMEM_SHARED((kpad, 128), _f32),
            pltpu.VMEM_SHARED((kd, 128), _f32),
            pltpu.VMEM_SHARED((kd,), _f32),
        ],
        name=f"gat_edge_{kpad}",
    )
    return kern(comps, compd, cnt, als, ald, m, h, znum, zden)


def _dec_agg_body(y_hbm, comps, compd, cnt_hbm, zrow_hbm, agg_out,
                  sbufw, dbufw, rowbuf, cntv, ysh, aggsh):
    c = lax.axis_index("c")
    s = lax.axis_index("s")
    wid = _wid()
    rows_y = KP3 // 16
    rows_t = KD3 // 16

    pltpu.sync_copy(cnt_hbm.at[pl.ds(wid * 128, 128)], cntv)
    pltpu.sync_copy(y_hbm.at[pl.ds(s * rows_y, rows_y)],
                    ysh.at[pl.ds(s * rows_y, rows_y)])
    pltpu.sync_copy(zrow_hbm, aggsh.at[pl.ds(s * rows_t, rows_t)])
    plsc.subcore_barrier()

    cb = cntv[pl.ds(0, 16)][0]
    segbase = wid * PTP

    def chunk_body(j, carry):
        @pl.when(j * 128 < cb)
        def _():
            pltpu.sync_copy(comps.at[pl.ds(segbase + j * 128, 128)], sbufw)
            pltpu.sync_copy(compd.at[pl.ds(segbase + j * 128, 128)], dbufw)
            for v in range(8):
                sv = sbufw[pl.ds(v * 16, 16)]
                dv = dbufw[pl.ds(v * 16, 16)]
                lanepos = j * 128 + v * 16 + lax.iota(_i32, 16)
                mvld = lanepos < cb
                sbufw[pl.ds(v * 16, 16)] = jnp.where(mvld, sv, 0)
                dbufw[pl.ds(v * 16, 16)] = jnp.where(mvld, dv, KP3)
            pltpu.sync_copy(ysh.at[sbufw], rowbuf)
            pltpu.sync_copy(rowbuf, aggsh.at[dbufw], add=True)

        return carry

    lax.fori_loop(0, NCH, chunk_body, 0)
    plsc.subcore_barrier()
    pltpu.sync_copy(aggsh.at[pl.ds(s * rows_t, rows_t)],
                    agg_out.at[pl.ds(c * KD3 + s * rows_t, rows_t)])


def _dec_agg(y, comps, compd, cnt):
    zrow = jnp.zeros((KD3 // 16, 128), _f32)
    kern = pl.kernel(
        _dec_agg_body,
        out_type=jax.ShapeDtypeStruct((NC * KD3, 128), _f32),
        mesh=_sc_mesh(),
        compiler_params=pltpu.CompilerParams(needs_layout_passes=False),
        scratch_types=[
            pltpu.VMEM((128,), _i32), pltpu.VMEM((128,), _i32),
            pltpu.VMEM((128, 128), _f32),
            pltpu.VMEM((128,), _i32),
            pltpu.VMEM_SHARED((KP3, 128), _f32),
            pltpu.VMEM_SHARED((KD3, 128), _f32),
        ],
        name="dec_agg",
    )
    return kern(y, comps, compd, cnt, zrow)


def _prep0_body(degT_ref, x4_ref, y_ref, dis_ref):
    degT = degT_ref[...]
    d = degT[:, 0:1] + degT[:, 1:2] + 1.0
    dis = lax.rsqrt(d)
    dis_ref[...] = dis
    y_ref[...] = dis * x4_ref[...]


def _prep0(degT, x4):
    return pl.pallas_call(
        _prep0_body,
        out_shape=(jax.ShapeDtypeStruct((NPAD0, 4), _f32),
                   jax.ShapeDtypeStruct((NPAD0, 1), _f32)),
    )(degT, x4)


def _gcn0_fin_body(agg_ref, x4_ref, dis_ref, W4_ref, b_ref, p_ref,
                   h_ref, score_ref):
    dis = dis_ref[...]
    agg = agg_ref[...]
    aggs = agg[0] + agg[1]
    pre = dis * aggs + (dis * dis) * x4_ref[...]
    h = jnp.dot(pre, W4_ref[...], preferred_element_type=_f32) + b_ref[...]
    h_ref[...] = h
    p = p_ref[...]
    pn = p * lax.rsqrt(jnp.sum(p * p))
    proj = jnp.dot(h, pn, preferred_element_type=_f32)
    rows = lax.broadcasted_iota(_i32, (NPAD0, 1), 0)
    score_ref[...] = jnp.where(rows < N0, jnp.tanh(proj), -1e30)


def _gcn0_fin(agg, x4, dis, W4, b, p):
    return pl.pallas_call(
        _gcn0_fin_body,
        out_shape=(jax.ShapeDtypeStruct((NPAD0, H), _f32),
                   jax.ShapeDtypeStruct((NPAD0, 1), _f32)),
    )(agg, x4, dis, W4, b, p)


def _rank_body(npad, scol_ref, srow_ref, out_ref):
    ib = pl.program_id(0)
    si = scol_ref[...]
    i_ids = ib * 512 + lax.broadcasted_iota(_i32, (512, 1), 0)

    def body(j, acc):
        sj = srow_ref[:, pl.ds(j * 512, 512)]
        j_ids = j * 512 + lax.broadcasted_iota(_i32, (1, 512), 1)
        gt = (sj > si).astype(_i32)
        eq = ((sj == si) & (j_ids < i_ids)).astype(_i32)
        return acc + jnp.sum(gt + eq, axis=1, keepdims=True)

    out_ref[...] = lax.fori_loop(0, npad // 512, body,
                                 jnp.zeros((512, 1), _i32))


def _rank(score_col, npad):
    srow = score_col.reshape(1, npad)
    return pl.pallas_call(
        functools.partial(_rank_body, npad),
        grid=(npad // 512,),
        in_specs=[pl.BlockSpec((512, 1), lambda i: (i, 0)),
                  pl.BlockSpec((1, npad), lambda i: (0, 0))],
        out_specs=pl.BlockSpec((512, 1), lambda i: (i, 0)),
        out_shape=jax.ShapeDtypeStruct((npad, 1), _i32),
    )(score_col, srow)


def _select_body(npad, k, rank_ref, srow_ref, h_ref, hsel_ref, vals_ref):
    rb = pl.program_id(0)
    r_ids = rb * 256 + lax.broadcasted_iota(_i32, (256, 1), 0)

    def body(ic, carry):
        acc, vacc = carry
        rk = rank_ref[:, pl.ds(ic * 512, 512)]
        sc = srow_ref[:, pl.ds(ic * 512, 512)]
        hit = (rk == r_ids) & (r_ids < k)
        P = jnp.where(hit, sc, 0.0)
        acc = acc + jnp.dot(P, h_ref[pl.ds(ic * 512, 512), :],
                            preferred_element_type=_f32)
        vacc = vacc + jnp.sum(P, axis=1, keepdims=True)
        return acc, vacc

    acc, vacc = lax.fori_loop(
        0, npad // 512, body,
        (jnp.zeros((256, H), _f32), jnp.zeros((256, 1), _f32)))
    hsel_ref[...] = acc
    vals_ref[...] = vacc


def _select(rank_col, score_col, h, npad, k, kpad):
    rrow = rank_col.reshape(1, npad)
    srow = score_col.reshape(1, npad)
    return pl.pallas_call(
        functools.partial(_select_body, npad, k),
        grid=(kpad // 256,),
        in_specs=[pl.BlockSpec((1, npad), lambda i: (0, 0)),
                  pl.BlockSpec((1, npad), lambda i: (0, 0)),
                  pl.BlockSpec((npad, H), lambda i: (0, 0))],
        out_specs=(pl.BlockSpec((256, H), lambda i: (i, 0)),
                   pl.BlockSpec((256, 1), lambda i: (i, 0))),
        out_shape=(jax.ShapeDtypeStruct((kpad, H), _f32),
                   jax.ShapeDtypeStruct((kpad, 1), _f32)),
    )(rrow, srow, h)


def _gat_dense_body(n, kpad, hsel_ref, W_ref, as_ref, ad_ref,
                    h_ref, als_ref, ald_ref, m_ref, ws_ref):
    rows = lax.broadcasted_iota(_i32, (kpad, 1), 0)
    h1 = hsel_ref[...]
    h = jnp.dot(h1, W_ref[...], preferred_element_type=_f32)
    h_ref[...] = h
    als = jnp.dot(h, as_ref[...], preferred_element_type=_f32)
    ald = jnp.dot(h, ad_ref[...], preferred_element_type=_f32)
    als_ref[...] = als
    ald_ref[...] = ald
    amax = jnp.max(jnp.where(rows < n, als, -3e38))
    pre_m = amax + ald
    m = jnp.where(pre_m > 0, pre_m, 0.2 * pre_m)
    m_ref[...] = m
    pre_s = als + ald
    es = jnp.where(pre_s > 0, pre_s, 0.2 * pre_s)
    ws_ref[...] = jnp.maximum(jnp.exp(es - m), 1e-30)


def _gat_dense(hsel, W, a_s, a_d, n, kpad):
    return pl.pallas_call(
        functools.partial(_gat_dense_body, n, kpad),
        out_shape=(jax.ShapeDtypeStruct((kpad, H), _f32),
                   jax.ShapeDtypeStruct((kpad, 1), _f32),
                   jax.ShapeDtypeStruct((kpad, 1), _f32),
                   jax.ShapeDtypeStruct((kpad, 1), _f32),
                   jax.ShapeDtypeStruct((kpad, 1), _f32)),
    )(hsel, W, a_s, a_d)


def _gat_fin_body(n, kpad, num_ref, den_ref, ws_ref, h_ref, b_ref, p_ref,
                  hout_ref, score_ref):
    ws = ws_ref[...]
    num_a = num_ref[...]
    den_a = den_ref[...]
    num = (num_a[0, :kpad, :] + num_a[1, :kpad, :] + ws * h_ref[...])
    den = (den_a[0, :kpad, :] + den_a[1, :kpad, :] + ws)
    out = num / den + b_ref[...]
    hout_ref[...] = out
    p = p_ref[...]
    pn = p * lax.rsqrt(jnp.sum(p * p))
    proj = jnp.dot(out, pn, preferred_element_type=_f32)
    rows = lax.broadcasted_iota(_i32, (kpad, 1), 0)
    score_ref[...] = jnp.where(rows < n, jnp.tanh(proj), -1e30)


def _gat_fin(num, den3, ws, h, b, p, n, kpad):
    return pl.pallas_call(
        functools.partial(_gat_fin_body, n, kpad),
        out_shape=(jax.ShapeDtypeStruct((kpad, H), _f32),
                   jax.ShapeDtypeStruct((kpad, 1), _f32)),
    )(num, den3, ws, h, b, p)


def _vae_body(degT_ref, hsel_ref, Wmu_ref, bmu_ref, Wlv_ref, blv_ref,
              Wld_ref, bld_ref, eps_ref,
              mu_ref, lv_ref, z_ref, y_ref, dis_ref):
    h5 = hsel_ref[...]
    mu = jnp.dot(h5, Wmu_ref[...], preferred_element_type=_f32) + bmu_ref[...]
    lv = jnp.dot(h5, Wlv_ref[...], preferred_element_type=_f32) + blv_ref[...]
    mu_ref[...] = mu
    lv_ref[...] = lv
    zlat = mu + eps_ref[...] * jnp.exp(0.5 * lv)
    z = jnp.dot(zlat, Wld_ref[...], preferred_element_type=_f32) + bld_ref[...]
    z_ref[...] = z
    degT = degT_ref[...]
    d = degT[:, 0:1] + degT[:, 1:2] + 2.0
    dis = lax.rsqrt(d)
    dis_ref[...] = dis
    y_ref[...] = dis * z


def _vae(degT, hsel, Wmu, bmu, Wlv, blv, Wld, bld, eps):
    return pl.pallas_call(
        _vae_body,
        out_shape=(jax.ShapeDtypeStruct((KP3, LAT), _f32),
                   jax.ShapeDtypeStruct((KP3, LAT), _f32),
                   jax.ShapeDtypeStruct((KP3, H), _f32),
                   jax.ShapeDtypeStruct((KP3, H), _f32),
                   jax.ShapeDtypeStruct((KP3, 1), _f32)),
    )(degT, hsel, Wmu, bmu, Wlv, blv, Wld, bld, eps)


def _dec_fin_body(agg_ref, z_ref, dis_ref, W_ref, b_ref, zn_ref, yn_ref):
    dis = dis_ref[...]
    agg = agg_ref[...]
    aggs = agg[0, :KP3, :] + agg[1, :KP3, :]
    pre = dis * aggs + 2.0 * (dis * dis) * z_ref[...]
    zn = jnp.dot(pre, W_ref[...], preferred_element_type=_f32) + b_ref[...]
    zn_ref[...] = zn
    yn_ref[...] = dis * zn


def _dec_fin(agg, z, dis, W, b):
    return pl.pallas_call(
        _dec_fin_body,
        out_shape=(jax.ShapeDtypeStruct((KP3, H), _f32),
                   jax.ShapeDtypeStruct((KP3, H), _f32)),
    )(agg, z, dis, W, b)


def kernel(x, edge_index, undirected_edge_index, batch, params):
    p = params
    src = edge_index[0]
    dst = edge_index[1]
    src2d = src.reshape(NSUB, 128)
    dst2d = dst.reshape(NSUB, 128)

    x4 = jnp.pad(x, ((0, NPAD0 - N0), (0, 1)))
    W4 = jnp.pad(p['W_e0'], ((0, 1), (0, 0)))

    deg, degdec, comps, compd, cnt = _edge_prep(src, dst)

    y0, dis0 = _prep0(deg.reshape(NC, NPAD0).T, x4)
    agg0 = _gcn0_agg(y0, src, dst).reshape(NC, NPAD0, 4)
    h0, score0 = _gcn0_fin(agg0, x4, dis0, W4, p['b_e0'].reshape(1, H),
                           p['p0'].reshape(H, 1))

    rk0 = _rank(score0, NPAD0)
    hsel1, _ = _select(rk0, score0, h0, NPAD0, K1, KP1)
    hg1, als1, ald1, m1, ws1 = _gat_dense(hsel1, p['W_e1'],
                                          p['a_src1'].reshape(H, 1),
                                          p['a_dst1'].reshape(H, 1), K1, KP1)
    num1, den1 = _gat_edge(KP1, KD1, (0, 1, 2), comps, compd, cnt,
                           als1.reshape(KP1), ald1.reshape(KP1),
                           m1.reshape(KP1), hg1)
    h1, score1 = _gat_fin(num1.reshape(NC, KD1, H),
                          den1.reshape(NC, KD1, 1), ws1, hg1,
                          p['b_e1'].reshape(1, H), p['p1'].reshape(H, 1),
                          K1, KP1)

    rk1 = _rank(score1, KP1)
    hsel2, _ = _select(rk1, score1, h1, KP1, K2, KP2)
    hg2, als2, ald2, m2, ws2 = _gat_dense(hsel2, p['W_e2'],
                                          p['a_src2'].reshape(H, 1),
                                          p['a_dst2'].reshape(H, 1), K2, KP2)
    num2, den2 = _gat_edge(KP2, KD2, (0, 1), comps, compd, cnt,
                           als2.reshape(KP2), ald2.reshape(KP2),
                           m2.reshape(KP2), hg2)
    h2, score2 = _gat_fin(num2.reshape(NC, KD2, H),
                          den2.reshape(NC, KD2, 1), ws2, hg2,
                          p['b_e2'].reshape(1, H), p['p2'].reshape(H, 1),
                          K2, KP2)

    rk2 = _rank(score2, KP2)
    hsel3, _ = _select(rk2, score2, h2, KP2, K3, KP3)
    eps = jax.random.normal(jax.random.key(42), (K3, LAT), _f32)
    eps = jnp.pad(eps, ((0, KP3 - K3), (0, 0)))
    degdecT = degdec.reshape(NC, KD3).T[:KP3, :]
    mu, lv, z, y, disd = _vae(degdecT, hsel3,
                              p['W_mu'], p['b_mu'].reshape(1, LAT),
                              p['W_lv'], p['b_lv'].reshape(1, LAT),
                              p['W_ld'], p['b_ld'].reshape(1, H), eps)

    for Wd, bd in [(p['W_d2'], p['b_d2']), (p['W_d1'], p['b_d1']),
                   (p['W_d0'], p['b_d0'])]:
        aggd = _dec_agg(y, comps, compd, cnt).reshape(NC, KD3, H)
        z, y = _dec_fin(aggd, z, disd, Wd, bd.reshape(1, H))

    return z[:K3], mu[:K3], lv[:K3]


# speedup vs baseline: 169.2427x; 1.0027x over previous
import functools

import jax
import jax.numpy as jnp
from jax import lax
from jax.experimental import pallas as pl
from jax.experimental.pallas import tpu as pltpu
from jax.experimental.pallas import tpu_sc as plsc

NC, NS = 2, 16
NW = NC * NS
N0, E, H, LAT = 10000, 640000, 128, 16
NPAD0 = 10240
PT = E // NW
ECH = 2560
NECH = E // ECH
PTP = 20608
NCH = PTP // 128
NSUB = E // 128
NBLK8 = E // (128 * 8)

K1, K2, K3 = 5000, 2500, 1250
KP1, KP2, KP3 = 5120, 2560, 1280
KD1, KD2, KD3 = 5248, 2688, 1408
CDUMP = 3 * NW * PTP

_f32 = jnp.float32
_i32 = jnp.int32


def _sc_mesh():
    return plsc.VectorSubcoreMesh(core_axis_name="c", subcore_axis_name="s")


def _wid():
    return lax.axis_index("s") * NC + lax.axis_index("c")


_SPD = NS * 3 * 2 * PTP


def _edge_prep_body(src_hbm, dst_hbm, one_hbm, z640_hbm,
                    deg_out, degdec_out, comps_out, compd_out, cnt_out,
                    sbuf, dbuf, pidx, pdidx, degidx, ddidx, ddval, one128,
                    cntv, sem, degsh, ddsh, spcomp):
    c = lax.axis_index("c")
    s = lax.axis_index("s")
    wid = _wid()

    pltpu.sync_copy(one_hbm, one128)
    pltpu.sync_copy(z640_hbm, degsh.at[pl.ds(s * 640, 640)])

    @pl.when(s < KD3 // 128)
    def _():
        pltpu.sync_copy(z640_hbm.at[pl.ds(0, 128)], ddsh.at[pl.ds(s * 128, 128)])

    plsc.subcore_barrier()

    lanes16 = lax.iota(_i32, 16)
    baseA = (s * 3 + 0) * 2 * PTP
    baseB = (s * 3 + 1) * 2 * PTP
    baseC = (s * 3 + 2) * 2 * PTP
    dumpbase = _SPD + s * 128

    def chunk_body(i, offs):
        ch = wid + NW * i

        def do_chunk(offs_in):
            d1 = pltpu.async_copy(src_hbm.at[pl.ds(ch * ECH, ECH)], sbuf, sem)
            d2 = pltpu.async_copy(dst_hbm.at[pl.ds(ch * ECH, ECH)], dbuf, sem)
            d1.wait()
            d2.wait()

            def grp_body(g, offs2):
                offA, offB, offC = offs2
                for v in range(8):
                    sv = sbuf[pl.ds(g * 128 + v * 16, 16)]
                    dv = dbuf[pl.ds(g * 128 + v * 16, 16)]
                    mx = jnp.maximum(sv, dv)
                    mA = mx < K3
                    mB = (mx >= K3) & (mx < K2)
                    mC = (mx >= K2) & (mx < K1)
                    valid = mx < K1
                    csA = plsc.cumsum(mA.astype(_i32))
                    csB = plsc.cumsum(mB.astype(_i32))
                    csC = plsc.cumsum(mC.astype(_i32))
                    dumpv = dumpbase + v * 16 + lanes16
                    posS = jnp.where(
                        mA, baseA + offA + csA - 1,
                        jnp.where(mB, baseB + offB + csB - 1,
                                  jnp.where(mC, baseC + offC + csC - 1,
                                            dumpv)))
                    posD = posS + jnp.where(valid, PTP, 0)
                    pidx[pl.ds(v * 16, 16)] = posS
                    pdidx[pl.ds(v * 16, 16)] = posD
                    degidx[pl.ds(v * 16, 16)] = dv
                    ddidx[pl.ds(v * 16, 16)] = jnp.where(mA, dv, KP3)
                    ddval[pl.ds(v * 16, 16)] = jnp.where(mA, 1.0, 0.0).astype(_f32)
                    offA = offA + csA[15]
                    offB = offB + csB[15]
                    offC = offC + csC[15]
                vals_s = sbuf.at[pl.ds(g * 128, 128)]
                vals_d = dbuf.at[pl.ds(g * 128, 128)]
                descs = [
                    pltpu.async_copy(vals_s, spcomp.at[pidx], sem),
                    pltpu.async_copy(vals_d, spcomp.at[pdidx], sem),
                    pltpu.async_copy(one128, degsh.at[degidx], sem, add=True),
                    pltpu.async_copy(ddval, ddsh.at[ddidx], sem, add=True),
                ]
                for d in descs:
                    d.wait()
                return offA, offB, offC

            return lax.fori_loop(0, ECH // 128, grp_body, offs_in)

        return lax.cond(ch < NECH, do_chunk, lambda o: o, offs)

    z = jnp.int32(0)
    offA, offB, offC = lax.fori_loop(0, (NECH + NW - 1) // NW, chunk_body,
                                     (z, z, z))

    for b, base in enumerate((baseA, baseB, baseC)):
        seg = (b * NW + wid) * PTP
        pltpu.sync_copy(spcomp.at[pl.ds(base, PTP)],
                        comps_out.at[pl.ds(seg, PTP)])
        pltpu.sync_copy(spcomp.at[pl.ds(base + PTP, PTP)],
                        compd_out.at[pl.ds(seg, PTP)])

    lanes = lax.iota(_i32, 16)
    cvec = jnp.zeros((16,), _i32)
    for b, off_b in enumerate((offA, offB, offC)):
        cvec = jnp.where(lanes == b, jnp.zeros((16,), _i32) + off_b, cvec)
    cntv[pl.ds(0, 16)] = cvec
    for t in range(1, 8):
        cntv[pl.ds(t * 16, 16)] = jnp.zeros((16,), _i32)
    pltpu.sync_copy(cntv, cnt_out.at[pl.ds(wid * 128, 128)])

    plsc.subcore_barrier()

    @pl.when(s == 0)
    def _():
        pltpu.sync_copy(degsh, deg_out.at[pl.ds(c * NPAD0, NPAD0)])
        pltpu.sync_copy(ddsh, degdec_out.at[pl.ds(c * KD3, KD3)])


def _edge_prep(src, dst):
    ones = jnp.ones((128,), _f32)
    z640 = jnp.zeros((640,), _f32)
    kern = pl.kernel(
        _edge_prep_body,
        out_type=(
            jax.ShapeDtypeStruct((NC * NPAD0,), _f32),
            jax.ShapeDtypeStruct((NC * KD3,), _f32),
            jax.ShapeDtypeStruct((CDUMP,), _i32),
            jax.ShapeDtypeStruct((CDUMP,), _i32),
            jax.ShapeDtypeStruct((NW * 128,), _i32),
        ),
        mesh=_sc_mesh(),
        compiler_params=pltpu.CompilerParams(needs_layout_passes=False),
        scratch_types=[
            pltpu.VMEM((ECH,), _i32), pltpu.VMEM((ECH,), _i32),
            pltpu.VMEM((128,), _i32), pltpu.VMEM((128,), _i32),
            pltpu.VMEM((128,), _i32), pltpu.VMEM((128,), _i32),
            pltpu.VMEM((128,), _f32), pltpu.VMEM((128,), _f32),
            pltpu.VMEM((128,), _i32),
            pltpu.SemaphoreType.DMA,
            pltpu.VMEM_SHARED((NPAD0,), _f32),
            pltpu.VMEM_SHARED((KD3,), _f32),
            pltpu.VMEM_SHARED((_SPD + NS * 128,), _i32),
        ],
        name="edge_prep",
    )
    return kern(src, dst, ones, z640)


def _gcn0_agg_body(y_hbm, src_hbm, dst_hbm, z640_hbm, agg_out,
                   si128, di128, gidx0, gidx1, gidx2, gidx3,
                   sidx0, sidx1, sidx2, sidx3,
                   gbuf0, gbuf1, gbuf2, gbuf3, semg, sems, aggsh):
    c = lax.axis_index("c")
    s = lax.axis_index("s")
    wid = _wid()
    gidx = (gidx0, gidx1, gidx2, gidx3)
    sidx = (sidx0, sidx1, sidx2, sidx3)
    gbuf = (gbuf0, gbuf1, gbuf2, gbuf3)

    for q in range(4):
        pltpu.sync_copy(z640_hbm, aggsh.at[pl.ds(s * 2560 + q * 640, 640)])
    plsc.subcore_barrier()

    def blk_body(j, carry):
        sub = wid + NW * j

        @pl.when(sub < NSUB)
        def _():
            base = sub * 128
            d1 = pltpu.async_copy(src_hbm.at[pl.ds(base, 128)], si128, semg)
            d2 = pltpu.async_copy(dst_hbm.at[pl.ds(base, 128)], di128, semg)
            d1.wait()
            d2.wait()
            for cc in range(4):
                for v in range(8):
                    sv = si128[pl.ds(v * 16, 16)]
                    dv = di128[pl.ds(v * 16, 16)]
                    gidx[cc][pl.ds(v * 16, 16)] = sv * 4 + cc
                    sidx[cc][pl.ds(v * 16, 16)] = dv * 4 + cc
            descs = [pltpu.async_copy(y_hbm.at[gidx[cc]], gbuf[cc], semg)
                     for cc in range(4)]
            for d in descs:
                d.wait()
            descs = [pltpu.async_copy(gbuf[cc], aggsh.at[sidx[cc]], sems,
                                      add=True)
                     for cc in range(4)]
            for d in descs:
                d.wait()

        return carry

    lax.fori_loop(0, (NSUB + NW - 1) // NW, blk_body, 0)
    plsc.subcore_barrier()
    pltpu.sync_copy(aggsh.at[pl.ds(s * 2560, 2560)],
                    agg_out.at[pl.ds(c * NPAD0 * 4 + s * 2560, 2560)])


def _gcn0_agg(y, src, dst):
    z640 = jnp.zeros((640,), _f32)
    kern = pl.kernel(
        _gcn0_agg_body,
        out_type=jax.ShapeDtypeStruct((NC * NPAD0 * 4,), _f32),
        mesh=_sc_mesh(),
        compiler_params=pltpu.CompilerParams(needs_layout_passes=False),
        scratch_types=[
            pltpu.VMEM((128,), _i32), pltpu.VMEM((128,), _i32),
            pltpu.VMEM((128,), _i32), pltpu.VMEM((128,), _i32),
            pltpu.VMEM((128,), _i32), pltpu.VMEM((128,), _i32),
            pltpu.VMEM((128,), _i32), pltpu.VMEM((128,), _i32),
            pltpu.VMEM((128,), _i32), pltpu.VMEM((128,), _i32),
            pltpu.VMEM((128,), _f32), pltpu.VMEM((128,), _f32),
            pltpu.VMEM((128,), _f32), pltpu.VMEM((128,), _f32),
            pltpu.SemaphoreType.DMA, pltpu.SemaphoreType.DMA,
            pltpu.VMEM_SHARED((NPAD0 * 4,), _f32),
        ],
        name="gcn0_agg",
    )
    return kern(y.reshape(NPAD0 * 4), src, dst, z640)


def _gat_edge_body(kpad, kd, buckets,
                   comps, compd, cnt_hbm, als_hbm, ald_hbm, m_hbm, h_hbm,
                   znum_hbm, zden_hbm,
                   num_out, den_out,
                   alsv, aldv, mv, sbufw, dbufw, wbuf, rowbuf,
                   cntv, hsh, numsh, densh):
    c = lax.axis_index("c")
    s = lax.axis_index("s")
    wid = _wid()
    rows_h = kpad // 16
    rows_t = kd // 16
    dump = kpad

    nchd = kd // 128
    pltpu.sync_copy(als_hbm, alsv)
    pltpu.sync_copy(ald_hbm, aldv)
    pltpu.sync_copy(m_hbm, mv)
    pltpu.sync_copy(cnt_hbm.at[pl.ds(wid * 128, 128)], cntv)
    pltpu.sync_copy(h_hbm.at[pl.ds(s * rows_h, rows_h)],
                    hsh.at[pl.ds(s * rows_h, rows_h)])
    pltpu.sync_copy(znum_hbm, numsh.at[pl.ds(s * rows_t, rows_t)])
    for i in range((nchd + NS - 1) // NS):
        chunk = s + NS * i

        @pl.when(chunk < nchd)
        def _(chunk=chunk):
            pltpu.sync_copy(zden_hbm, densh.at[pl.ds(chunk * 128, 128)])

    plsc.subcore_barrier()

    cntvec = cntv[pl.ds(0, 16)]
    for b in buckets:
        cb = cntvec[b]
        segbase = (b * NW + wid) * PTP

        def chunk_body(j, carry, cb=cb, segbase=segbase):
            @pl.when(j * 128 < cb)
            def _():
                pltpu.sync_copy(comps.at[pl.ds(segbase + j * 128, 128)], sbufw)
                pltpu.sync_copy(compd.at[pl.ds(segbase + j * 128, 128)], dbufw)
                for v in range(8):
                    sv = sbufw[pl.ds(v * 16, 16)]
                    dv = dbufw[pl.ds(v * 16, 16)]
                    lanepos = j * 128 + v * 16 + lax.iota(_i32, 16)
                    mvld = lanepos < cb
                    s_c = jnp.where(mvld, sv, 0)
                    d_t = jnp.where(mvld, dv, 0)
                    a1 = plsc.load_gather(alsv, [s_c])
                    a2 = plsc.load_gather(aldv, [d_t])
                    mm = plsc.load_gather(mv, [d_t])
                    e = a1 + a2
                    e = jnp.where(e > 0, e, 0.2 * e)
                    w = jnp.where(mvld, jnp.exp(e - mm), 0.0)
                    wbuf[pl.ds(v * 16, 16)] = w
                    sbufw[pl.ds(v * 16, 16)] = s_c
                    dbufw[pl.ds(v * 16, 16)] = jnp.where(mvld, dv, dump)
                pltpu.sync_copy(wbuf.at[pl.ds(0, 128)], densh.at[dbufw],
                                add=True)
                pltpu.sync_copy(hsh.at[sbufw], rowbuf)

                def scale_row(r, carry2):
                    wr = wbuf[pl.ds(r, 16)][0]
                    for cc in range(8):
                        rowbuf[r, pl.ds(cc * 16, 16)] = (
                            rowbuf[r, pl.ds(cc * 16, 16)] * wr)
                    return carry2

                lax.fori_loop(0, 128, scale_row, 0)
                pltpu.sync_copy(rowbuf, numsh.at[dbufw], add=True)

            return carry

        lax.fori_loop(0, NCH, chunk_body, 0)

    plsc.subcore_barrier()
    pltpu.sync_copy(numsh.at[pl.ds(s * rows_t, rows_t)],
                    num_out.at[pl.ds(c * kd + s * rows_t, rows_t)])
    for i in range((nchd + NS - 1) // NS):
        chunk = s + NS * i

        @pl.when(chunk < nchd)
        def _(chunk=chunk):
            pltpu.sync_copy(densh.at[pl.ds(chunk * 128, 128)],
                            den_out.at[pl.ds(c * kd + chunk * 128, 128)])


def _gat_edge(kpad, kd, buckets, comps, compd, cnt, als, ald, m, h):
    rows_t = kd // 16
    znum = jnp.zeros((rows_t, 128), _f32)
    zden = jnp.zeros((128,), _f32)
    kern = pl.kernel(
        functools.partial(_gat_edge_body, kpad, kd, buckets),
        out_type=(
            jax.ShapeDtypeStruct((NC * kd, 128), _f32),
            jax.ShapeDtypeStruct((NC * kd,), _f32),
        ),
        mesh=_sc_mesh(),
        compiler_params=pltpu.CompilerParams(needs_layout_passes=False),
        scratch_types=[
            pltpu.VMEM((kpad,), _f32), pltpu.VMEM((kpad,), _f32),
            pltpu.VMEM((kpad,), _f32),
            pltpu.VMEM((128,), _i32), pltpu.VMEM((128,), _i32),
            pltpu.VMEM((144,), _f32), pltpu.VMEM((128, 128), _f32),
            pltpu.VMEM((128,), _i32),
            pltpu.VMEM_SHARED((kpad, 128), _f32),
            pltpu.VMEM_SHARED((kd, 128), _f32),
            pltpu.VMEM_SHARED((kd,), _f32),
        ],
        name=f"gat_edge_{kpad}",
    )
    return kern(comps, compd, cnt, als, ald, m, h, znum, zden)


def _dec_agg_body(y_hbm, comps, compd, cnt_hbm, zrow_hbm, agg_out,
                  sbufw, dbufw, rowbuf, cntv, ysh, aggsh):
    c = lax.axis_index("c")
    s = lax.axis_index("s")
    wid = _wid()
    rows_y = KP3 // 16
    rows_t = KD3 // 16

    pltpu.sync_copy(cnt_hbm.at[pl.ds(wid * 128, 128)], cntv)
    pltpu.sync_copy(y_hbm.at[pl.ds(s * rows_y, rows_y)],
                    ysh.at[pl.ds(s * rows_y, rows_y)])
    pltpu.sync_copy(zrow_hbm, aggsh.at[pl.ds(s * rows_t, rows_t)])
    plsc.subcore_barrier()

    cb = cntv[pl.ds(0, 16)][0]
    segbase = wid * PTP

    def chunk_body(j, carry):
        @pl.when(j * 128 < cb)
        def _():
            pltpu.sync_copy(comps.at[pl.ds(segbase + j * 128, 128)], sbufw)
            pltpu.sync_copy(compd.at[pl.ds(segbase + j * 128, 128)], dbufw)
            for v in range(8):
                sv = sbufw[pl.ds(v * 16, 16)]
                dv = dbufw[pl.ds(v * 16, 16)]
                lanepos = j * 128 + v * 16 + lax.iota(_i32, 16)
                mvld = lanepos < cb
                sbufw[pl.ds(v * 16, 16)] = jnp.where(mvld, sv, 0)
                dbufw[pl.ds(v * 16, 16)] = jnp.where(mvld, dv, KP3)
            pltpu.sync_copy(ysh.at[sbufw], rowbuf)
            pltpu.sync_copy(rowbuf, aggsh.at[dbufw], add=True)

        return carry

    lax.fori_loop(0, NCH, chunk_body, 0)
    plsc.subcore_barrier()
    pltpu.sync_copy(aggsh.at[pl.ds(s * rows_t, rows_t)],
                    agg_out.at[pl.ds(c * KD3 + s * rows_t, rows_t)])


def _dec_agg(y, comps, compd, cnt):
    zrow = jnp.zeros((KD3 // 16, 128), _f32)
    kern = pl.kernel(
        _dec_agg_body,
        out_type=jax.ShapeDtypeStruct((NC * KD3, 128), _f32),
        mesh=_sc_mesh(),
        compiler_params=pltpu.CompilerParams(needs_layout_passes=False),
        scratch_types=[
            pltpu.VMEM((128,), _i32), pltpu.VMEM((128,), _i32),
            pltpu.VMEM((128, 128), _f32),
            pltpu.VMEM((128,), _i32),
            pltpu.VMEM_SHARED((KP3, 128), _f32),
            pltpu.VMEM_SHARED((KD3, 128), _f32),
        ],
        name="dec_agg",
    )
    return kern(y, comps, compd, cnt, zrow)


def _prep0_body(degT_ref, x4_ref, y_ref, dis_ref):
    degT = degT_ref[...]
    d = degT[:, 0:1] + degT[:, 1:2] + 1.0
    dis = lax.rsqrt(d)
    dis_ref[...] = dis
    y_ref[...] = dis * x4_ref[...]


def _prep0(degT, x4):
    return pl.pallas_call(
        _prep0_body,
        out_shape=(jax.ShapeDtypeStruct((NPAD0, 4), _f32),
                   jax.ShapeDtypeStruct((NPAD0, 1), _f32)),
    )(degT, x4)


def _gcn0_fin_body(agg_ref, x4_ref, dis_ref, W4_ref, b_ref, p_ref,
                   h_ref, score_ref):
    dis = dis_ref[...]
    agg = agg_ref[...]
    aggs = agg[0] + agg[1]
    pre = dis * aggs + (dis * dis) * x4_ref[...]
    h = jnp.dot(pre, W4_ref[...], preferred_element_type=_f32) + b_ref[...]
    h_ref[...] = h
    p = p_ref[...]
    pn = p * lax.rsqrt(jnp.sum(p * p))
    proj = jnp.dot(h, pn, preferred_element_type=_f32)
    rows = lax.broadcasted_iota(_i32, (NPAD0, 1), 0)
    score_ref[...] = jnp.where(rows < N0, jnp.tanh(proj), -1e30)


def _gcn0_fin(agg, x4, dis, W4, b, p):
    return pl.pallas_call(
        _gcn0_fin_body,
        out_shape=(jax.ShapeDtypeStruct((NPAD0, H), _f32),
                   jax.ShapeDtypeStruct((NPAD0, 1), _f32)),
    )(agg, x4, dis, W4, b, p)


def _rank_body(npad, scol_ref, srow_ref, out_ref):
    ib = pl.program_id(0)
    si = scol_ref[...]
    i_ids = ib * 512 + lax.broadcasted_iota(_i32, (512, 1), 0)

    def body(j, acc):
        sj = srow_ref[:, pl.ds(j * 512, 512)]
        j_ids = j * 512 + lax.broadcasted_iota(_i32, (1, 512), 1)
        gt = (sj > si).astype(_i32)
        eq = ((sj == si) & (j_ids < i_ids)).astype(_i32)
        return acc + jnp.sum(gt + eq, axis=1, keepdims=True)

    out_ref[...] = lax.fori_loop(0, npad // 512, body,
                                 jnp.zeros((512, 1), _i32))


def _rank(score_col, npad):
    srow = score_col.reshape(1, npad)
    return pl.pallas_call(
        functools.partial(_rank_body, npad),
        grid=(npad // 512,),
        in_specs=[pl.BlockSpec((512, 1), lambda i: (i, 0)),
                  pl.BlockSpec((1, npad), lambda i: (0, 0))],
        out_specs=pl.BlockSpec((512, 1), lambda i: (i, 0)),
        out_shape=jax.ShapeDtypeStruct((npad, 1), _i32),
    )(score_col, srow)


def _select_body(npad, k, rank_ref, srow_ref, h_ref, hsel_ref, vals_ref):
    rb = pl.program_id(0)
    r_ids = rb * 256 + lax.broadcasted_iota(_i32, (256, 1), 0)

    def body(ic, carry):
        acc, vacc = carry
        rk = rank_ref[:, pl.ds(ic * 512, 512)]
        sc = srow_ref[:, pl.ds(ic * 512, 512)]
        hit = (rk == r_ids) & (r_ids < k)
        P = jnp.where(hit, sc, 0.0)
        acc = acc + jnp.dot(P, h_ref[pl.ds(ic * 512, 512), :],
                            preferred_element_type=_f32)
        vacc = vacc + jnp.sum(P, axis=1, keepdims=True)
        return acc, vacc

    acc, vacc = lax.fori_loop(
        0, npad // 512, body,
        (jnp.zeros((256, H), _f32), jnp.zeros((256, 1), _f32)))
    hsel_ref[...] = acc
    vals_ref[...] = vacc


def _select(rank_col, score_col, h, npad, k, kpad):
    rrow = rank_col.reshape(1, npad)
    srow = score_col.reshape(1, npad)
    return pl.pallas_call(
        functools.partial(_select_body, npad, k),
        grid=(kpad // 256,),
        in_specs=[pl.BlockSpec((1, npad), lambda i: (0, 0)),
                  pl.BlockSpec((1, npad), lambda i: (0, 0)),
                  pl.BlockSpec((npad, H), lambda i: (0, 0))],
        out_specs=(pl.BlockSpec((256, H), lambda i: (i, 0)),
                   pl.BlockSpec((256, 1), lambda i: (i, 0))),
        out_shape=(jax.ShapeDtypeStruct((kpad, H), _f32),
                   jax.ShapeDtypeStruct((kpad, 1), _f32)),
    )(rrow, srow, h)


def _gat_dense_body(n, kpad, hsel_ref, W_ref, as_ref, ad_ref,
                    h_ref, als_ref, ald_ref, m_ref, ws_ref):
    rows = lax.broadcasted_iota(_i32, (kpad, 1), 0)
    h1 = hsel_ref[...]
    h = jnp.dot(h1, W_ref[...], preferred_element_type=_f32)
    h_ref[...] = h
    als = jnp.dot(h, as_ref[...], preferred_element_type=_f32)
    ald = jnp.dot(h, ad_ref[...], preferred_element_type=_f32)
    als_ref[...] = als
    ald_ref[...] = ald
    amax = jnp.max(jnp.where(rows < n, als, -3e38))
    pre_m = amax + ald
    m = jnp.where(pre_m > 0, pre_m, 0.2 * pre_m)
    m_ref[...] = m
    pre_s = als + ald
    es = jnp.where(pre_s > 0, pre_s, 0.2 * pre_s)
    ws_ref[...] = jnp.maximum(jnp.exp(es - m), 1e-30)


def _gat_dense(hsel, W, a_s, a_d, n, kpad):
    return pl.pallas_call(
        functools.partial(_gat_dense_body, n, kpad),
        out_shape=(jax.ShapeDtypeStruct((kpad, H), _f32),
                   jax.ShapeDtypeStruct((kpad, 1), _f32),
                   jax.ShapeDtypeStruct((kpad, 1), _f32),
                   jax.ShapeDtypeStruct((kpad, 1), _f32),
                   jax.ShapeDtypeStruct((kpad, 1), _f32)),
    )(hsel, W, a_s, a_d)


def _gat_fin_body(n, kpad, num_ref, den_ref, ws_ref, h_ref, b_ref, p_ref,
                  hout_ref, score_ref):
    ws = ws_ref[...]
    num_a = num_ref[...]
    den_a = den_ref[...]
    num = (num_a[0, :kpad, :] + num_a[1, :kpad, :] + ws * h_ref[...])
    den = (den_a[0, :kpad, :] + den_a[1, :kpad, :] + ws)
    out = num / den + b_ref[...]
    hout_ref[...] = out
    p = p_ref[...]
    pn = p * lax.rsqrt(jnp.sum(p * p))
    proj = jnp.dot(out, pn, preferred_element_type=_f32)
    rows = lax.broadcasted_iota(_i32, (kpad, 1), 0)
    score_ref[...] = jnp.where(rows < n, jnp.tanh(proj), -1e30)


def _gat_fin(num, den3, ws, h, b, p, n, kpad):
    return pl.pallas_call(
        functools.partial(_gat_fin_body, n, kpad),
        out_shape=(jax.ShapeDtypeStruct((kpad, H), _f32),
                   jax.ShapeDtypeStruct((kpad, 1), _f32)),
    )(num, den3, ws, h, b, p)


def _vae_body(degT_ref, hsel_ref, Wmu_ref, bmu_ref, Wlv_ref, blv_ref,
              Wld_ref, bld_ref, eps_ref,
              mu_ref, lv_ref, z_ref, y_ref, dis_ref):
    h5 = hsel_ref[...]
    mu = jnp.dot(h5, Wmu_ref[...], preferred_element_type=_f32) + bmu_ref[...]
    lv = jnp.dot(h5, Wlv_ref[...], preferred_element_type=_f32) + blv_ref[...]
    mu_ref[...] = mu
    lv_ref[...] = lv
    zlat = mu + eps_ref[...] * jnp.exp(0.5 * lv)
    z = jnp.dot(zlat, Wld_ref[...], preferred_element_type=_f32) + bld_ref[...]
    z_ref[...] = z
    degT = degT_ref[...]
    d = degT[:, 0:1] + degT[:, 1:2] + 2.0
    dis = lax.rsqrt(d)
    dis_ref[...] = dis
    y_ref[...] = dis * z


def _vae(degT, hsel, Wmu, bmu, Wlv, blv, Wld, bld, eps):
    return pl.pallas_call(
        _vae_body,
        out_shape=(jax.ShapeDtypeStruct((KP3, LAT), _f32),
                   jax.ShapeDtypeStruct((KP3, LAT), _f32),
                   jax.ShapeDtypeStruct((KP3, H), _f32),
                   jax.ShapeDtypeStruct((KP3, H), _f32),
                   jax.ShapeDtypeStruct((KP3, 1), _f32)),
    )(degT, hsel, Wmu, bmu, Wlv, blv, Wld, bld, eps)


def _dec_fin_body(agg_ref, z_ref, dis_ref, W_ref, b_ref, zn_ref, yn_ref):
    dis = dis_ref[...]
    agg = agg_ref[...]
    aggs = agg[0, :KP3, :] + agg[1, :KP3, :]
    pre = dis * aggs + 2.0 * (dis * dis) * z_ref[...]
    zn = jnp.dot(pre, W_ref[...], preferred_element_type=_f32) + b_ref[...]
    zn_ref[...] = zn
    yn_ref[...] = dis * zn


def _dec_fin(agg, z, dis, W, b):
    return pl.pallas_call(
        _dec_fin_body,
        out_shape=(jax.ShapeDtypeStruct((KP3, H), _f32),
                   jax.ShapeDtypeStruct((KP3, H), _f32)),
    )(agg, z, dis, W, b)


def kernel(x, edge_index, undirected_edge_index, batch, params):
    p = params
    src = edge_index[0]
    dst = edge_index[1]
    src2d = src.reshape(NSUB, 128)
    dst2d = dst.reshape(NSUB, 128)

    x4 = jnp.pad(x, ((0, NPAD0 - N0), (0, 1)))
    W4 = jnp.pad(p['W_e0'], ((0, 1), (0, 0)))

    deg, degdec, comps, compd, cnt = _edge_prep(src, dst)

    y0, dis0 = _prep0(deg.reshape(NC, NPAD0).T, x4)
    agg0 = _gcn0_agg(y0, src, dst).reshape(NC, NPAD0, 4)
    h0, score0 = _gcn0_fin(agg0, x4, dis0, W4, p['b_e0'].reshape(1, H),
                           p['p0'].reshape(H, 1))

    rk0 = _rank(score0, NPAD0)
    hsel1, _ = _select(rk0, score0, h0, NPAD0, K1, KP1)
    hg1, als1, ald1, m1, ws1 = _gat_dense(hsel1, p['W_e1'],
                                          p['a_src1'].reshape(H, 1),
                                          p['a_dst1'].reshape(H, 1), K1, KP1)
    num1, den1 = _gat_edge(KP1, KD1, (0, 1, 2), comps, compd, cnt,
                           als1.reshape(KP1), ald1.reshape(KP1),
                           m1.reshape(KP1), hg1)
    h1, score1 = _gat_fin(num1.reshape(NC, KD1, H),
                          den1.reshape(NC, KD1, 1), ws1, hg1,
                          p['b_e1'].reshape(1, H), p['p1'].reshape(H, 1),
                          K1, KP1)

    rk1 = _rank(score1, KP1)
    hsel2, _ = _select(rk1, score1, h1, KP1, K2, KP2)
    hg2, als2, ald2, m2, ws2 = _gat_dense(hsel2, p['W_e2'],
                                          p['a_src2'].reshape(H, 1),
                                          p['a_dst2'].reshape(H, 1), K2, KP2)
    num2, den2 = _gat_edge(KP2, KD2, (0, 1), comps, compd, cnt,
                           als2.reshape(KP2), ald2.reshape(KP2),
                           m2.reshape(KP2), hg2)
    h2, score2 = _gat_fin(num2.reshape(NC, KD2, H),
                          den2.reshape(NC, KD2, 1), ws2, hg2,
                          p['b_e2'].reshape(1, H), p['p2'].reshape(H, 1),
                          K2, KP2)

    rk2 = _rank(score2, KP2)
    hsel3, _ = _select(rk2, score2, h2, KP2, K3, KP3)
    eps = jax.random.normal(jax.random.key(42), (K3, LAT), _f32)
    eps = jnp.pad(eps, ((0, KP3 - K3), (0, 0)))
    degdecT = degdec.reshape(NC, KD3).T[:KP3, :]
    mu, lv, z, y, disd = _vae(degdecT, hsel3,
                              p['W_mu'], p['b_mu'].reshape(1, LAT),
                              p['W_lv'], p['b_lv'].reshape(1, LAT),
                              p['W_ld'], p['b_ld'].reshape(1, H), eps)

    for Wd, bd in [(p['W_d2'], p['b_d2']), (p['W_d1'], p['b_d1']),
                   (p['W_d0'], p['b_d0'])]:
        aggd = _dec_agg(y, comps, compd, cnt).reshape(NC, KD3, H)
        z, y = _dec_fin(aggd, z, disd, Wd, bd.reshape(1, H))

    return z[:K3], mu[:K3], lv[:K3]


# speedup vs baseline: 180.4105x; 1.0660x over previous
import functools

import jax
import jax.numpy as jnp
from jax import lax
from jax.experimental import pallas as pl
from jax.experimental.pallas import tpu as pltpu
from jax.experimental.pallas import tpu_sc as plsc

NC, NS = 2, 16
NW = NC * NS
N0, E, H, LAT = 10000, 640000, 128, 16
NPAD0 = 10240
PT = E // NW
ECH = 2560
NECH = E // ECH
PTP = 20608
NCH = PTP // 128
NSUB = E // 128
NBLK8 = E // (128 * 8)

K1, K2, K3 = 5000, 2500, 1250
KP1, KP2, KP3 = 5120, 2560, 1280
KD1, KD2, KD3 = 5248, 2688, 1408
CDUMP = 3 * NW * PTP

_f32 = jnp.float32
_i32 = jnp.int32


def _sc_mesh():
    return plsc.VectorSubcoreMesh(core_axis_name="c", subcore_axis_name="s")


def _wid():
    return lax.axis_index("s") * NC + lax.axis_index("c")


_SPD = NS * 3 * 2 * PTP


def _edge_prep_body(src_hbm, dst_hbm, one_hbm, z640_hbm,
                    deg_out, degdec_out, comps_out, compd_out, cnt_out,
                    sbuf, dbuf, pidx, pdidx, degidx, ddidx, ddval, one128,
                    cntv, sem, degsh, ddsh, spcomp):
    c = lax.axis_index("c")
    s = lax.axis_index("s")
    wid = _wid()

    pltpu.sync_copy(one_hbm, one128)
    pltpu.sync_copy(z640_hbm, degsh.at[pl.ds(s * 640, 640)])

    @pl.when(s < KD3 // 128)
    def _():
        pltpu.sync_copy(z640_hbm.at[pl.ds(0, 128)], ddsh.at[pl.ds(s * 128, 128)])

    plsc.subcore_barrier()

    lanes16 = lax.iota(_i32, 16)
    baseA = (s * 3 + 0) * 2 * PTP
    baseB = (s * 3 + 1) * 2 * PTP
    baseC = (s * 3 + 2) * 2 * PTP
    dumpbase = _SPD + s * 128

    def chunk_body(i, offs):
        ch = wid + NW * i

        def do_chunk(offs_in):
            d1 = pltpu.async_copy(src_hbm.at[pl.ds(ch * ECH, ECH)], sbuf, sem)
            d2 = pltpu.async_copy(dst_hbm.at[pl.ds(ch * ECH, ECH)], dbuf, sem)
            d1.wait()
            d2.wait()

            def grp_body(g, offs2):
                offA, offB, offC = offs2
                for v in range(8):
                    sv = sbuf[pl.ds(g * 128 + v * 16, 16)]
                    dv = dbuf[pl.ds(g * 128 + v * 16, 16)]
                    mx = jnp.maximum(sv, dv)
                    mA = mx < K3
                    mB = (mx >= K3) & (mx < K2)
                    mC = (mx >= K2) & (mx < K1)
                    valid = mx < K1
                    csA = plsc.cumsum(mA.astype(_i32))
                    csB = plsc.cumsum(mB.astype(_i32))
                    csC = plsc.cumsum(mC.astype(_i32))
                    dumpv = dumpbase + v * 16 + lanes16
                    posS = jnp.where(
                        mA, baseA + offA + csA - 1,
                        jnp.where(mB, baseB + offB + csB - 1,
                                  jnp.where(mC, baseC + offC + csC - 1,
                                            dumpv)))
                    posD = posS + jnp.where(valid, PTP, 0)
                    pidx[pl.ds(v * 16, 16)] = posS
                    pdidx[pl.ds(v * 16, 16)] = posD
                    degidx[pl.ds(v * 16, 16)] = dv
                    ddidx[pl.ds(v * 16, 16)] = jnp.where(mA, dv, KP3)
                    ddval[pl.ds(v * 16, 16)] = jnp.where(mA, 1.0, 0.0).astype(_f32)
                    offA = offA + csA[15]
                    offB = offB + csB[15]
                    offC = offC + csC[15]
                vals_s = sbuf.at[pl.ds(g * 128, 128)]
                vals_d = dbuf.at[pl.ds(g * 128, 128)]
                descs = [
                    pltpu.async_copy(vals_s, spcomp.at[pidx], sem),
                    pltpu.async_copy(vals_d, spcomp.at[pdidx], sem),
                    pltpu.async_copy(one128, degsh.at[degidx], sem, add=True),
                    pltpu.async_copy(ddval, ddsh.at[ddidx], sem, add=True),
                ]
                for d in descs:
                    d.wait()
                return offA, offB, offC

            return lax.fori_loop(0, ECH // 128, grp_body, offs_in)

        return lax.cond(ch < NECH, do_chunk, lambda o: o, offs)

    z = jnp.int32(0)
    offA, offB, offC = lax.fori_loop(0, (NECH + NW - 1) // NW, chunk_body,
                                     (z, z, z))

    for b, base in enumerate((baseA, baseB, baseC)):
        seg = (b * NW + wid) * PTP
        pltpu.sync_copy(spcomp.at[pl.ds(base, PTP)],
                        comps_out.at[pl.ds(seg, PTP)])
        pltpu.sync_copy(spcomp.at[pl.ds(base + PTP, PTP)],
                        compd_out.at[pl.ds(seg, PTP)])

    lanes = lax.iota(_i32, 16)
    cvec = jnp.zeros((16,), _i32)
    for b, off_b in enumerate((offA, offB, offC)):
        cvec = jnp.where(lanes == b, jnp.zeros((16,), _i32) + off_b, cvec)
    cntv[pl.ds(0, 16)] = cvec
    for t in range(1, 8):
        cntv[pl.ds(t * 16, 16)] = jnp.zeros((16,), _i32)
    pltpu.sync_copy(cntv, cnt_out.at[pl.ds(wid * 128, 128)])

    plsc.subcore_barrier()

    @pl.when(s == 0)
    def _():
        pltpu.sync_copy(degsh, deg_out.at[pl.ds(c * NPAD0, NPAD0)])
        pltpu.sync_copy(ddsh, degdec_out.at[pl.ds(c * KD3, KD3)])


def _edge_prep(src, dst):
    ones = jnp.ones((128,), _f32)
    z640 = jnp.zeros((640,), _f32)
    kern = pl.kernel(
        _edge_prep_body,
        out_type=(
            jax.ShapeDtypeStruct((NC * NPAD0,), _f32),
            jax.ShapeDtypeStruct((NC * KD3,), _f32),
            jax.ShapeDtypeStruct((CDUMP,), _i32),
            jax.ShapeDtypeStruct((CDUMP,), _i32),
            jax.ShapeDtypeStruct((NW * 128,), _i32),
        ),
        mesh=_sc_mesh(),
        compiler_params=pltpu.CompilerParams(needs_layout_passes=False),
        scratch_types=[
            pltpu.VMEM((ECH,), _i32), pltpu.VMEM((ECH,), _i32),
            pltpu.VMEM((128,), _i32), pltpu.VMEM((128,), _i32),
            pltpu.VMEM((128,), _i32), pltpu.VMEM((128,), _i32),
            pltpu.VMEM((128,), _f32), pltpu.VMEM((128,), _f32),
            pltpu.VMEM((128,), _i32),
            pltpu.SemaphoreType.DMA,
            pltpu.VMEM_SHARED((NPAD0,), _f32),
            pltpu.VMEM_SHARED((KD3,), _f32),
            pltpu.VMEM_SHARED((_SPD + NS * 128,), _i32),
        ],
        name="edge_prep",
    )
    return kern(src, dst, ones, z640)


def _gcn0_agg_body(y_hbm, src_hbm, dst_hbm, z640_hbm, agg_out,
                   si256, di256, *rest):
    gidx = rest[0:8]
    sidx = rest[8:16]
    gbuf = rest[16:24]
    semg, sems, aggsh = rest[24], rest[25], rest[26]
    c = lax.axis_index("c")
    s = lax.axis_index("s")
    wid = _wid()

    for q in range(4):
        pltpu.sync_copy(z640_hbm, aggsh.at[pl.ds(s * 2560 + q * 640, 640)])
    plsc.subcore_barrier()

    def blk_body(j, carry):
        blk = wid + NW * j

        @pl.when(blk < NSUB // 2)
        def _():
            base = blk * 256
            d1 = pltpu.async_copy(src_hbm.at[pl.ds(base, 256)], si256, semg)
            d2 = pltpu.async_copy(dst_hbm.at[pl.ds(base, 256)], di256, semg)
            d1.wait()
            d2.wait()
            for half in range(2):
                for cc in range(4):
                    k8 = half * 4 + cc
                    for v in range(8):
                        sv = si256[pl.ds(half * 128 + v * 16, 16)]
                        dv = di256[pl.ds(half * 128 + v * 16, 16)]
                        gidx[k8][pl.ds(v * 16, 16)] = sv * 4 + cc
                        sidx[k8][pl.ds(v * 16, 16)] = dv * 4 + cc
            descs = [pltpu.async_copy(y_hbm.at[gidx[k8]], gbuf[k8], semg)
                     for k8 in range(8)]
            for d in descs:
                d.wait()
            descs = [pltpu.async_copy(gbuf[k8], aggsh.at[sidx[k8]], sems,
                                      add=True)
                     for k8 in range(8)]
            for d in descs:
                d.wait()

        return carry

    lax.fori_loop(0, (NSUB // 2 + NW - 1) // NW, blk_body, 0)
    plsc.subcore_barrier()
    pltpu.sync_copy(aggsh.at[pl.ds(s * 2560, 2560)],
                    agg_out.at[pl.ds(c * NPAD0 * 4 + s * 2560, 2560)])


def _gcn0_agg(y, src, dst):
    z640 = jnp.zeros((640,), _f32)
    kern = pl.kernel(
        _gcn0_agg_body,
        out_type=jax.ShapeDtypeStruct((NC * NPAD0 * 4,), _f32),
        mesh=_sc_mesh(),
        compiler_params=pltpu.CompilerParams(needs_layout_passes=False),
        scratch_types=(
            [pltpu.VMEM((256,), _i32), pltpu.VMEM((256,), _i32)]
            + [pltpu.VMEM((128,), _i32)] * 16
            + [pltpu.VMEM((128,), _f32)] * 8
            + [pltpu.SemaphoreType.DMA, pltpu.SemaphoreType.DMA,
               pltpu.VMEM_SHARED((NPAD0 * 4,), _f32)]
        ),
        name="gcn0_agg",
    )
    return kern(y.reshape(NPAD0 * 4), src, dst, z640)


def _gat_edge_body(kpad, kd, buckets,
                   comps, compd, cnt_hbm, als_hbm, ald_hbm, m_hbm, h_hbm,
                   znum_hbm, zden_hbm,
                   num_out, den_out,
                   alsv, aldv, mv, sbufw, dbufw, wbuf, rowbuf,
                   cntv, hsh, numsh, densh):
    c = lax.axis_index("c")
    s = lax.axis_index("s")
    wid = _wid()
    rows_h = kpad // 16
    rows_t = kd // 16
    dump = kpad

    nchd = kd // 128
    pltpu.sync_copy(als_hbm, alsv)
    pltpu.sync_copy(ald_hbm, aldv)
    pltpu.sync_copy(m_hbm, mv)
    pltpu.sync_copy(cnt_hbm.at[pl.ds(wid * 128, 128)], cntv)
    pltpu.sync_copy(h_hbm.at[pl.ds(s * rows_h, rows_h)],
                    hsh.at[pl.ds(s * rows_h, rows_h)])
    pltpu.sync_copy(znum_hbm, numsh.at[pl.ds(s * rows_t, rows_t)])
    for i in range((nchd + NS - 1) // NS):
        chunk = s + NS * i

        @pl.when(chunk < nchd)
        def _(chunk=chunk):
            pltpu.sync_copy(zden_hbm, densh.at[pl.ds(chunk * 128, 128)])

    plsc.subcore_barrier()

    cntvec = cntv[pl.ds(0, 16)]
    for b in buckets:
        cb = cntvec[b]
        segbase = (b * NW + wid) * PTP

        def chunk_body(j, carry, cb=cb, segbase=segbase):
            @pl.when(j * 128 < cb)
            def _():
                pltpu.sync_copy(comps.at[pl.ds(segbase + j * 128, 128)], sbufw)
                pltpu.sync_copy(compd.at[pl.ds(segbase + j * 128, 128)], dbufw)
                for v in range(8):
                    sv = sbufw[pl.ds(v * 16, 16)]
                    dv = dbufw[pl.ds(v * 16, 16)]
                    lanepos = j * 128 + v * 16 + lax.iota(_i32, 16)
                    mvld = lanepos < cb
                    s_c = jnp.where(mvld, sv, 0)
                    d_t = jnp.where(mvld, dv, 0)
                    a1 = plsc.load_gather(alsv, [s_c])
                    a2 = plsc.load_gather(aldv, [d_t])
                    mm = plsc.load_gather(mv, [d_t])
                    e = a1 + a2
                    e = jnp.where(e > 0, e, 0.2 * e)
                    w = jnp.where(mvld, jnp.exp(e - mm), 0.0)
                    wbuf[pl.ds(v * 16, 16)] = w
                    sbufw[pl.ds(v * 16, 16)] = s_c
                    dbufw[pl.ds(v * 16, 16)] = jnp.where(mvld, dv, dump)
                pltpu.sync_copy(wbuf.at[pl.ds(0, 128)], densh.at[dbufw],
                                add=True)
                pltpu.sync_copy(hsh.at[sbufw], rowbuf)

                def scale_row(r, carry2):
                    wr = wbuf[pl.ds(r, 16)][0]
                    for cc in range(8):
                        rowbuf[r, pl.ds(cc * 16, 16)] = (
                            rowbuf[r, pl.ds(cc * 16, 16)] * wr)
                    return carry2

                lax.fori_loop(0, 128, scale_row, 0)
                pltpu.sync_copy(rowbuf, numsh.at[dbufw], add=True)

            return carry

        lax.fori_loop(0, NCH, chunk_body, 0)

    plsc.subcore_barrier()
    pltpu.sync_copy(numsh.at[pl.ds(s * rows_t, rows_t)],
                    num_out.at[pl.ds(c * kd + s * rows_t, rows_t)])
    for i in range((nchd + NS - 1) // NS):
        chunk = s + NS * i

        @pl.when(chunk < nchd)
        def _(chunk=chunk):
            pltpu.sync_copy(densh.at[pl.ds(chunk * 128, 128)],
                            den_out.at[pl.ds(c * kd + chunk * 128, 128)])


def _gat_edge(kpad, kd, buckets, comps, compd, cnt, als, ald, m, h):
    rows_t = kd // 16
    znum = jnp.zeros((rows_t, 128), _f32)
    zden = jnp.zeros((128,), _f32)
    kern = pl.kernel(
        functools.partial(_gat_edge_body, kpad, kd, buckets),
        out_type=(
            jax.ShapeDtypeStruct((NC * kd, 128), _f32),
            jax.ShapeDtypeStruct((NC * kd,), _f32),
        ),
        mesh=_sc_mesh(),
        compiler_params=pltpu.CompilerParams(needs_layout_passes=False),
        scratch_types=[
            pltpu.VMEM((kpad,), _f32), pltpu.VMEM((kpad,), _f32),
            pltpu.VMEM((kpad,), _f32),
            pltpu.VMEM((128,), _i32), pltpu.VMEM((128,), _i32),
            pltpu.VMEM((144,), _f32), pltpu.VMEM((128, 128), _f32),
            pltpu.VMEM((128,), _i32),
            pltpu.VMEM_SHARED((kpad, 128), _f32),
            pltpu.VMEM_SHARED((kd, 128), _f32),
            pltpu.VMEM_SHARED((kd,), _f32),
        ],
        name=f"gat_edge_{kpad}",
    )
    return kern(comps, compd, cnt, als, ald, m, h, znum, zden)


def _dec_agg_body(y_hbm, comps, compd, cnt_hbm, zrow_hbm, agg_out,
                  sbufw, dbufw, rowbuf, cntv, ysh, aggsh):
    c = lax.axis_index("c")
    s = lax.axis_index("s")
    wid = _wid()
    rows_y = KP3 // 16
    rows_t = KD3 // 16

    pltpu.sync_copy(cnt_hbm.at[pl.ds(wid * 128, 128)], cntv)
    pltpu.sync_copy(y_hbm.at[pl.ds(s * rows_y, rows_y)],
                    ysh.at[pl.ds(s * rows_y, rows_y)])
    pltpu.sync_copy(zrow_hbm, aggsh.at[pl.ds(s * rows_t, rows_t)])
    plsc.subcore_barrier()

    cb = cntv[pl.ds(0, 16)][0]
    segbase = wid * PTP

    def chunk_body(j, carry):
        @pl.when(j * 128 < cb)
        def _():
            pltpu.sync_copy(comps.at[pl.ds(segbase + j * 128, 128)], sbufw)
            pltpu.sync_copy(compd.at[pl.ds(segbase + j * 128, 128)], dbufw)
            for v in range(8):
                sv = sbufw[pl.ds(v * 16, 16)]
                dv = dbufw[pl.ds(v * 16, 16)]
                lanepos = j * 128 + v * 16 + lax.iota(_i32, 16)
                mvld = lanepos < cb
                sbufw[pl.ds(v * 16, 16)] = jnp.where(mvld, sv, 0)
                dbufw[pl.ds(v * 16, 16)] = jnp.where(mvld, dv, KP3)
            pltpu.sync_copy(ysh.at[sbufw], rowbuf)
            pltpu.sync_copy(rowbuf, aggsh.at[dbufw], add=True)

        return carry

    lax.fori_loop(0, NCH, chunk_body, 0)
    plsc.subcore_barrier()
    pltpu.sync_copy(aggsh.at[pl.ds(s * rows_t, rows_t)],
                    agg_out.at[pl.ds(c * KD3 + s * rows_t, rows_t)])


def _dec_agg(y, comps, compd, cnt):
    zrow = jnp.zeros((KD3 // 16, 128), _f32)
    kern = pl.kernel(
        _dec_agg_body,
        out_type=jax.ShapeDtypeStruct((NC * KD3, 128), _f32),
        mesh=_sc_mesh(),
        compiler_params=pltpu.CompilerParams(needs_layout_passes=False),
        scratch_types=[
            pltpu.VMEM((128,), _i32), pltpu.VMEM((128,), _i32),
            pltpu.VMEM((128, 128), _f32),
            pltpu.VMEM((128,), _i32),
            pltpu.VMEM_SHARED((KP3, 128), _f32),
            pltpu.VMEM_SHARED((KD3, 128), _f32),
        ],
        name="dec_agg",
    )
    return kern(y, comps, compd, cnt, zrow)


def _prep0_body(degT_ref, x4_ref, y_ref, dis_ref):
    degT = degT_ref[...]
    d = degT[:, 0:1] + degT[:, 1:2] + 1.0
    dis = lax.rsqrt(d)
    dis_ref[...] = dis
    y_ref[...] = dis * x4_ref[...]


def _prep0(degT, x4):
    return pl.pallas_call(
        _prep0_body,
        out_shape=(jax.ShapeDtypeStruct((NPAD0, 4), _f32),
                   jax.ShapeDtypeStruct((NPAD0, 1), _f32)),
    )(degT, x4)


def _gcn0_fin_body(agg_ref, x4_ref, dis_ref, W4_ref, b_ref, p_ref,
                   h_ref, score_ref):
    dis = dis_ref[...]
    agg = agg_ref[...]
    aggs = agg[0] + agg[1]
    pre = dis * aggs + (dis * dis) * x4_ref[...]
    h = jnp.dot(pre, W4_ref[...], preferred_element_type=_f32) + b_ref[...]
    h_ref[...] = h
    p = p_ref[...]
    pn = p * lax.rsqrt(jnp.sum(p * p))
    proj = jnp.dot(h, pn, preferred_element_type=_f32)
    rows = lax.broadcasted_iota(_i32, (NPAD0, 1), 0)
    score_ref[...] = jnp.where(rows < N0, jnp.tanh(proj), -1e30)


def _gcn0_fin(agg, x4, dis, W4, b, p):
    return pl.pallas_call(
        _gcn0_fin_body,
        out_shape=(jax.ShapeDtypeStruct((NPAD0, H), _f32),
                   jax.ShapeDtypeStruct((NPAD0, 1), _f32)),
    )(agg, x4, dis, W4, b, p)


def _rank_body(npad, scol_ref, srow_ref, out_ref):
    ib = pl.program_id(0)
    si = scol_ref[...]
    i_ids = ib * 512 + lax.broadcasted_iota(_i32, (512, 1), 0)

    def body(j, acc):
        sj = srow_ref[:, pl.ds(j * 512, 512)]
        j_ids = j * 512 + lax.broadcasted_iota(_i32, (1, 512), 1)
        gt = (sj > si).astype(_i32)
        eq = ((sj == si) & (j_ids < i_ids)).astype(_i32)
        return acc + jnp.sum(gt + eq, axis=1, keepdims=True)

    out_ref[...] = lax.fori_loop(0, npad // 512, body,
                                 jnp.zeros((512, 1), _i32))


def _rank(score_col, npad):
    srow = score_col.reshape(1, npad)
    return pl.pallas_call(
        functools.partial(_rank_body, npad),
        grid=(npad // 512,),
        in_specs=[pl.BlockSpec((512, 1), lambda i: (i, 0)),
                  pl.BlockSpec((1, npad), lambda i: (0, 0))],
        out_specs=pl.BlockSpec((512, 1), lambda i: (i, 0)),
        out_shape=jax.ShapeDtypeStruct((npad, 1), _i32),
    )(score_col, srow)


def _select_body(npad, k, rank_ref, srow_ref, h_ref, hsel_ref, vals_ref):
    rb = pl.program_id(0)
    r_ids = rb * 256 + lax.broadcasted_iota(_i32, (256, 1), 0)

    def body(ic, carry):
        acc, vacc = carry
        rk = rank_ref[:, pl.ds(ic * 512, 512)]
        sc = srow_ref[:, pl.ds(ic * 512, 512)]
        hit = (rk == r_ids) & (r_ids < k)
        P = jnp.where(hit, sc, 0.0)
        acc = acc + jnp.dot(P, h_ref[pl.ds(ic * 512, 512), :],
                            preferred_element_type=_f32)
        vacc = vacc + jnp.sum(P, axis=1, keepdims=True)
        return acc, vacc

    acc, vacc = lax.fori_loop(
        0, npad // 512, body,
        (jnp.zeros((256, H), _f32), jnp.zeros((256, 1), _f32)))
    hsel_ref[...] = acc
    vals_ref[...] = vacc


def _select(rank_col, score_col, h, npad, k, kpad):
    rrow = rank_col.reshape(1, npad)
    srow = score_col.reshape(1, npad)
    return pl.pallas_call(
        functools.partial(_select_body, npad, k),
        grid=(kpad // 256,),
        in_specs=[pl.BlockSpec((1, npad), lambda i: (0, 0)),
                  pl.BlockSpec((1, npad), lambda i: (0, 0)),
                  pl.BlockSpec((npad, H), lambda i: (0, 0))],
        out_specs=(pl.BlockSpec((256, H), lambda i: (i, 0)),
                   pl.BlockSpec((256, 1), lambda i: (i, 0))),
        out_shape=(jax.ShapeDtypeStruct((kpad, H), _f32),
                   jax.ShapeDtypeStruct((kpad, 1), _f32)),
    )(rrow, srow, h)


def _gat_dense_body(n, kpad, hsel_ref, W_ref, as_ref, ad_ref,
                    h_ref, als_ref, ald_ref, m_ref, ws_ref):
    rows = lax.broadcasted_iota(_i32, (kpad, 1), 0)
    h1 = hsel_ref[...]
    h = jnp.dot(h1, W_ref[...], preferred_element_type=_f32)
    h_ref[...] = h
    als = jnp.dot(h, as_ref[...], preferred_element_type=_f32)
    ald = jnp.dot(h, ad_ref[...], preferred_element_type=_f32)
    als_ref[...] = als
    ald_ref[...] = ald
    amax = jnp.max(jnp.where(rows < n, als, -3e38))
    pre_m = amax + ald
    m = jnp.where(pre_m > 0, pre_m, 0.2 * pre_m)
    m_ref[...] = m
    pre_s = als + ald
    es = jnp.where(pre_s > 0, pre_s, 0.2 * pre_s)
    ws_ref[...] = jnp.maximum(jnp.exp(es - m), 1e-30)


def _gat_dense(hsel, W, a_s, a_d, n, kpad):
    return pl.pallas_call(
        functools.partial(_gat_dense_body, n, kpad),
        out_shape=(jax.ShapeDtypeStruct((kpad, H), _f32),
                   jax.ShapeDtypeStruct((kpad, 1), _f32),
                   jax.ShapeDtypeStruct((kpad, 1), _f32),
                   jax.ShapeDtypeStruct((kpad, 1), _f32),
                   jax.ShapeDtypeStruct((kpad, 1), _f32)),
    )(hsel, W, a_s, a_d)


def _gat_fin_body(n, kpad, num_ref, den_ref, ws_ref, h_ref, b_ref, p_ref,
                  hout_ref, score_ref):
    ws = ws_ref[...]
    num_a = num_ref[...]
    den_a = den_ref[...]
    num = (num_a[0, :kpad, :] + num_a[1, :kpad, :] + ws * h_ref[...])
    den = (den_a[0, :kpad, :] + den_a[1, :kpad, :] + ws)
    out = num / den + b_ref[...]
    hout_ref[...] = out
    p = p_ref[...]
    pn = p * lax.rsqrt(jnp.sum(p * p))
    proj = jnp.dot(out, pn, preferred_element_type=_f32)
    rows = lax.broadcasted_iota(_i32, (kpad, 1), 0)
    score_ref[...] = jnp.where(rows < n, jnp.tanh(proj), -1e30)


def _gat_fin(num, den3, ws, h, b, p, n, kpad):
    return pl.pallas_call(
        functools.partial(_gat_fin_body, n, kpad),
        out_shape=(jax.ShapeDtypeStruct((kpad, H), _f32),
                   jax.ShapeDtypeStruct((kpad, 1), _f32)),
    )(num, den3, ws, h, b, p)


def _vae_body(degT_ref, hsel_ref, Wmu_ref, bmu_ref, Wlv_ref, blv_ref,
              Wld_ref, bld_ref, eps_ref,
              mu_ref, lv_ref, z_ref, y_ref, dis_ref):
    h5 = hsel_ref[...]
    mu = jnp.dot(h5, Wmu_ref[...], preferred_element_type=_f32) + bmu_ref[...]
    lv = jnp.dot(h5, Wlv_ref[...], preferred_element_type=_f32) + blv_ref[...]
    mu_ref[...] = mu
    lv_ref[...] = lv
    zlat = mu + eps_ref[...] * jnp.exp(0.5 * lv)
    z = jnp.dot(zlat, Wld_ref[...], preferred_element_type=_f32) + bld_ref[...]
    z_ref[...] = z
    degT = degT_ref[...]
    d = degT[:, 0:1] + degT[:, 1:2] + 2.0
    dis = lax.rsqrt(d)
    dis_ref[...] = dis
    y_ref[...] = dis * z


def _vae(degT, hsel, Wmu, bmu, Wlv, blv, Wld, bld, eps):
    return pl.pallas_call(
        _vae_body,
        out_shape=(jax.ShapeDtypeStruct((KP3, LAT), _f32),
                   jax.ShapeDtypeStruct((KP3, LAT), _f32),
                   jax.ShapeDtypeStruct((KP3, H), _f32),
                   jax.ShapeDtypeStruct((KP3, H), _f32),
                   jax.ShapeDtypeStruct((KP3, 1), _f32)),
    )(degT, hsel, Wmu, bmu, Wlv, blv, Wld, bld, eps)


def _dec_fin_body(agg_ref, z_ref, dis_ref, W_ref, b_ref, zn_ref, yn_ref):
    dis = dis_ref[...]
    agg = agg_ref[...]
    aggs = agg[0, :KP3, :] + agg[1, :KP3, :]
    pre = dis * aggs + 2.0 * (dis * dis) * z_ref[...]
    zn = jnp.dot(pre, W_ref[...], preferred_element_type=_f32) + b_ref[...]
    zn_ref[...] = zn
    yn_ref[...] = dis * zn


def _dec_fin(agg, z, dis, W, b):
    return pl.pallas_call(
        _dec_fin_body,
        out_shape=(jax.ShapeDtypeStruct((KP3, H), _f32),
                   jax.ShapeDtypeStruct((KP3, H), _f32)),
    )(agg, z, dis, W, b)


def kernel(x, edge_index, undirected_edge_index, batch, params):
    p = params
    src = edge_index[0]
    dst = edge_index[1]
    src2d = src.reshape(NSUB, 128)
    dst2d = dst.reshape(NSUB, 128)

    x4 = jnp.pad(x, ((0, NPAD0 - N0), (0, 1)))
    W4 = jnp.pad(p['W_e0'], ((0, 1), (0, 0)))

    deg, degdec, comps, compd, cnt = _edge_prep(src, dst)

    y0, dis0 = _prep0(deg.reshape(NC, NPAD0).T, x4)
    agg0 = _gcn0_agg(y0, src, dst).reshape(NC, NPAD0, 4)
    h0, score0 = _gcn0_fin(agg0, x4, dis0, W4, p['b_e0'].reshape(1, H),
                           p['p0'].reshape(H, 1))

    rk0 = _rank(score0, NPAD0)
    hsel1, _ = _select(rk0, score0, h0, NPAD0, K1, KP1)
    hg1, als1, ald1, m1, ws1 = _gat_dense(hsel1, p['W_e1'],
                                          p['a_src1'].reshape(H, 1),
                                          p['a_dst1'].reshape(H, 1), K1, KP1)
    num1, den1 = _gat_edge(KP1, KD1, (0, 1, 2), comps, compd, cnt,
                           als1.reshape(KP1), ald1.reshape(KP1),
                           m1.reshape(KP1), hg1)
    h1, score1 = _gat_fin(num1.reshape(NC, KD1, H),
                          den1.reshape(NC, KD1, 1), ws1, hg1,
                          p['b_e1'].reshape(1, H), p['p1'].reshape(H, 1),
                          K1, KP1)

    rk1 = _rank(score1, KP1)
    hsel2, _ = _select(rk1, score1, h1, KP1, K2, KP2)
    hg2, als2, ald2, m2, ws2 = _gat_dense(hsel2, p['W_e2'],
                                          p['a_src2'].reshape(H, 1),
                                          p['a_dst2'].reshape(H, 1), K2, KP2)
    num2, den2 = _gat_edge(KP2, KD2, (0, 1), comps, compd, cnt,
                           als2.reshape(KP2), ald2.reshape(KP2),
                           m2.reshape(KP2), hg2)
    h2, score2 = _gat_fin(num2.reshape(NC, KD2, H),
                          den2.reshape(NC, KD2, 1), ws2, hg2,
                          p['b_e2'].reshape(1, H), p['p2'].reshape(H, 1),
                          K2, KP2)

    rk2 = _rank(score2, KP2)
    hsel3, _ = _select(rk2, score2, h2, KP2, K3, KP3)
    eps = jax.random.normal(jax.random.key(42), (K3, LAT), _f32)
    eps = jnp.pad(eps, ((0, KP3 - K3), (0, 0)))
    degdecT = degdec.reshape(NC, KD3).T[:KP3, :]
    mu, lv, z, y, disd = _vae(degdecT, hsel3,
                              p['W_mu'], p['b_mu'].reshape(1, LAT),
                              p['W_lv'], p['b_lv'].reshape(1, LAT),
                              p['W_ld'], p['b_ld'].reshape(1, H), eps)

    for Wd, bd in [(p['W_d2'], p['b_d2']), (p['W_d1'], p['b_d1']),
                   (p['W_d0'], p['b_d0'])]:
        aggd = _dec_agg(y, comps, compd, cnt).reshape(NC, KD3, H)
        z, y = _dec_fin(aggd, z, disd, Wd, bd.reshape(1, H))

    return z[:K3], mu[:K3], lv[:K3]
